# Initial kernel scaffold; baseline (speedup 1.0000x reference)
import jax
import jax.numpy as jnp
from jax.experimental import pallas as pl


def kernel(user_ids, item_ids, content_features, edge_index, user_emb_table, item_emb_table, W_content, b_content, W1, att_src1, att_dst1, b1, W2, att_src2, att_dst2, b2, Wp1, bp1, Wp2, bp2):
    raise NotImplementedError("write your pallas kernel here")


# speedup vs baseline: 61.3596x; 61.3596x over previous
import functools

import jax
import jax.numpy as jnp
from jax import lax
from jax.experimental import pallas as pl
from jax.experimental.pallas import tpu as pltpu
from jax.experimental.pallas import tpu_sc as plsc

N = 50000
NU = 25000
B = 16384
E = 800000
XROWS = N + 16
EPS = 1e-16

_MESH = dict(core_axis_name="c", subcore_axis_name="s", num_cores=2,
             num_subcores=16)
_SC_PARAMS = pltpu.CompilerParams(needs_layout_passes=False,
                                  use_tc_tiling_on_sc=False)


def _tc_content(content, WcT, bc):
  r = 2048

  def body(c_ref, w_ref, b_ref, o_ref):
    o_ref[...] = (
        jnp.dot(c_ref[...], w_ref[...], preferred_element_type=jnp.float32)
        + b_ref[...])

  return pl.pallas_call(
      body,
      grid=(B // r,),
      in_specs=[
          pl.BlockSpec((r, 128), lambda i: (i, 0)),
          pl.BlockSpec((128, 32), lambda i: (0, 0)),
          pl.BlockSpec((1, 32), lambda i: (0, 0)),
      ],
      out_specs=pl.BlockSpec((r, 32), lambda i: (i, 0)),
      out_shape=jax.ShapeDtypeStruct((B, 32), jnp.float32),
  )(content, WcT, bc)


def _tc_layer1_prep(xL, xR, Wst, att_s, att_d):
  r = 2000
  nb = N // r
  oc = 32

  def body(xl_ref, xr_ref, w_ref, as_ref, ad_ref,
           hh_ref, ahs_ref, ahd_ref, inum_ref, iden_ref):
    cidx = pl.program_id(0)
    h = (jnp.dot(xl_ref[...], w_ref[0, :32, :],
                 preferred_element_type=jnp.float32)
         + jnp.dot(xr_ref[...], w_ref[0, 32:, :],
                   preferred_element_type=jnp.float32))
    asr = jnp.where(cidx == 0, as_ref[0:1, :], as_ref[1:2, :])
    adr = jnp.where(cidx == 0, ad_ref[0:1, :], ad_ref[1:2, :])
    a_s = jnp.sum(h * asr, axis=1, keepdims=True)
    a_d = jnp.sum(h * adr, axis=1, keepdims=True)
    al = a_s + a_d
    exs = jnp.exp(jnp.maximum(al, 0.2 * al))
    hh_ref[...] = h
    ahs_ref[...] = a_s
    ahd_ref[...] = a_d
    inum_ref[...] = h * exs
    iden_ref[...] = exs

  out_shapes = [
      jax.ShapeDtypeStruct((2 * N, oc), jnp.float32),
      jax.ShapeDtypeStruct((2 * N, 1), jnp.float32),
      jax.ShapeDtypeStruct((2 * N, 1), jnp.float32),
      jax.ShapeDtypeStruct((2 * N, oc), jnp.float32),
      jax.ShapeDtypeStruct((2 * N, 1), jnp.float32),
  ]
  return pl.pallas_call(
      body,
      grid=(2, nb),
      in_specs=[
          pl.BlockSpec((r, 32), lambda c, i: (i, 0)),
          pl.BlockSpec((r, 32), lambda c, i: (i, 0)),
          pl.BlockSpec((1, 64, oc), lambda c, i: (c, 0, 0)),
          pl.BlockSpec((2, oc), lambda c, i: (0, 0)),
          pl.BlockSpec((2, oc), lambda c, i: (0, 0)),
      ],
      out_specs=[
          pl.BlockSpec((r, oc), lambda c, i: (c * nb + i, 0)),
          pl.BlockSpec((r, 1), lambda c, i: (c * nb + i, 0)),
          pl.BlockSpec((r, 1), lambda c, i: (c * nb + i, 0)),
          pl.BlockSpec((r, oc), lambda c, i: (c * nb + i, 0)),
          pl.BlockSpec((r, 1), lambda c, i: (c * nb + i, 0)),
      ],
      out_shape=out_shapes,
  )(xL, xR, Wst, att_s, att_d)


def _tc_layer2_prep(num1, den1, b1v, W2st, W2full, att_s2, att_d2):
  r = 2000
  nb = N // r
  oc = 16

  def body(na_ref, nbb_ref, da_ref, db_ref, b1_ref, w_ref, wf_ref,
           as_ref, ad_ref, h2_ref, a2s_ref, a2d_ref, inum_ref, iden_ref):
    za = na_ref[...] / (da_ref[...] + EPS)
    zb = nbb_ref[...] / (db_ref[...] + EPS)
    z = jnp.concatenate([za, zb], axis=1) + b1_ref[...]
    x2 = jnp.where(z > 0, z, jnp.exp(z) - 1.0)
    h2 = jnp.dot(x2, w_ref[0], preferred_element_type=jnp.float32)
    ws = jnp.sum(wf_ref[...] * as_ref[...], axis=1, keepdims=True)
    wd = jnp.sum(wf_ref[...] * ad_ref[...], axis=1, keepdims=True)
    a2s = jnp.dot(x2, ws, preferred_element_type=jnp.float32)
    a2d = jnp.dot(x2, wd, preferred_element_type=jnp.float32)
    al = a2s + a2d
    exs = jnp.exp(jnp.maximum(al, 0.2 * al))
    h2_ref[...] = h2
    a2s_ref[...] = a2s
    a2d_ref[...] = a2d
    inum_ref[...] = h2 * exs
    iden_ref[...] = exs

  out_shapes = [
      jax.ShapeDtypeStruct((2 * N, oc), jnp.float32),
      jax.ShapeDtypeStruct((N, 1), jnp.float32),
      jax.ShapeDtypeStruct((N, 1), jnp.float32),
      jax.ShapeDtypeStruct((2 * N, oc), jnp.float32),
      jax.ShapeDtypeStruct((N, 1), jnp.float32),
  ]
  return pl.pallas_call(
      body,
      grid=(2, nb),
      in_specs=[
          pl.BlockSpec((r, 32), lambda c, i: (i, 0)),
          pl.BlockSpec((r, 32), lambda c, i: (nb + i, 0)),
          pl.BlockSpec((r, 1), lambda c, i: (i, 0)),
          pl.BlockSpec((r, 1), lambda c, i: (nb + i, 0)),
          pl.BlockSpec((1, 64), lambda c, i: (0, 0)),
          pl.BlockSpec((1, 64, oc), lambda c, i: (c, 0, 0)),
          pl.BlockSpec((64, 32), lambda c, i: (0, 0)),
          pl.BlockSpec((1, 32), lambda c, i: (0, 0)),
          pl.BlockSpec((1, 32), lambda c, i: (0, 0)),
      ],
      out_specs=[
          pl.BlockSpec((r, oc), lambda c, i: (c * nb + i, 0)),
          pl.BlockSpec((r, 1), lambda c, i: (i, 0)),
          pl.BlockSpec((r, 1), lambda c, i: (i, 0)),
          pl.BlockSpec((r, oc), lambda c, i: (c * nb + i, 0)),
          pl.BlockSpec((r, 1), lambda c, i: (i, 0)),
      ],
      out_shape=out_shapes,
  )(num1, num1, den1, den1, b1v, W2st, W2full, att_s2, att_d2)


def _tc_x3(num2, den2, b2v):
  r = 2000
  nb = N // r

  def body(na_ref, nbb_ref, d_ref, b_ref, o_ref):
    d = d_ref[...] + EPS
    o_ref[...] = jnp.concatenate(
        [na_ref[...] / d, nbb_ref[...] / d], axis=1) + b_ref[...]

  return pl.pallas_call(
      body,
      grid=(nb,),
      in_specs=[
          pl.BlockSpec((r, 16), lambda i: (i, 0)),
          pl.BlockSpec((r, 16), lambda i: (nb + i, 0)),
          pl.BlockSpec((r, 1), lambda i: (i, 0)),
          pl.BlockSpec((1, 32), lambda i: (0, 0)),
      ],
      out_specs=pl.BlockSpec((r, 32), lambda i: (i, 0)),
      out_shape=jax.ShapeDtypeStruct((N, 32), jnp.float32),
  )(num2, num2, den2, b2v)


def _tc_mlp(u_emb, ig, ug, Wp1T, bp1v, Wp2T, bp2v):
  r = 2048

  def body(ue_ref, ig_ref, ug_ref, w1_ref, b1_ref, w2_ref, b2_ref, o_ref):
    hdn = (jnp.dot(ue_ref[...], w1_ref[0:32, :],
                   preferred_element_type=jnp.float32)
           + jnp.dot(ig_ref[...], w1_ref[32:64, :],
                     preferred_element_type=jnp.float32)
           + jnp.dot(ug_ref[...], w1_ref[64:96, :],
                     preferred_element_type=jnp.float32)
           + b1_ref[...])
    hdn = jnp.maximum(hdn, 0.0)
    o_ref[...] = (jnp.dot(hdn, w2_ref[...],
                          preferred_element_type=jnp.float32) + b2_ref[...])

  return pl.pallas_call(
      body,
      grid=(B // r,),
      in_specs=[
          pl.BlockSpec((r, 32), lambda i: (i, 0)),
          pl.BlockSpec((r, 32), lambda i: (i, 0)),
          pl.BlockSpec((r, 32), lambda i: (i, 0)),
          pl.BlockSpec((96, 32), lambda i: (0, 0)),
          pl.BlockSpec((1, 32), lambda i: (0, 0)),
          pl.BlockSpec((32, 1), lambda i: (0, 0)),
          pl.BlockSpec((1, 1), lambda i: (0, 0)),
      ],
      out_specs=pl.BlockSpec((r, 1), lambda i: (i, 0)),
      out_shape=jax.ShapeDtypeStruct((B, 1), jnp.float32),
  )(u_emb, ig, ug, Wp1T, bp1v, Wp2T, bp2v)


def _sc_build_x(user_ids, item_ids, c_emb, utab, itab):
  CH = 512
  ZR = 392

  mesh = plsc.VectorSubcoreMesh(**_MESH)

  @functools.partial(
      pl.kernel,
      out_type=(
          jax.ShapeDtypeStruct((XROWS, 32), jnp.float32),
          jax.ShapeDtypeStruct((XROWS, 32), jnp.float32),
          jax.ShapeDtypeStruct((B, 32), jnp.float32),
      ),
      mesh=mesh,
      compiler_params=_SC_PARAMS,
      scratch_types=(
          pltpu.VMEM((ZR, 32), jnp.float32),
          pltpu.VMEM((CH,), jnp.int32),
          pltpu.VMEM((CH,), jnp.int32),
          pltpu.VMEM((CH, 32), jnp.float32),
          pltpu.VMEM((CH, 32), jnp.float32),
          pltpu.VMEM((B,), jnp.int32),
          pltpu.VMEM((NU,), jnp.int32),
          pltpu.VMEM_SHARED((B,), jnp.int32),
          pltpu.SemaphoreType.DMA,
      ),
  )
  def k(uids_hbm, iids_hbm, cemb_hbm, utab_hbm, itab_hbm,
        xl_hbm, xr_hbm, uemb_hbm,
        zb, idx_v, tgt_v, rows_v, ce_v, iid_v, tab_v, tgt_sp, sem):
    c = lax.axis_index("c")
    s = lax.axis_index("s")

    zero16 = jnp.zeros((16,), jnp.float32)

    @pl.loop(0, ZR)
    def _(j):
      zb[j, pl.ds(0, 16)] = zero16
      zb[j, pl.ds(16, 16)] = zero16

    row0 = c * NU + s * 1568

    def zcopy(nr, koff):
      sl = pl.ds(row0 + koff, nr)
      pltpu.sync_copy(zb.at[pl.ds(0, nr), :], xl_hbm.at[sl, :])
      pltpu.sync_copy(zb.at[pl.ds(0, nr), :], xr_hbm.at[sl, :])

    for kk in range(3):
      zcopy(ZR, kk * ZR)

    @pl.when(s < 15)
    def _():
      zcopy(ZR, 3 * ZR)

    @pl.when(jnp.logical_and(s == 15, c == 0))
    def _():
      zcopy(304, 3 * ZR)

    @pl.when(jnp.logical_and(s == 15, c == 1))
    def _():
      zcopy(320, 3 * ZR)

    @pl.when(jnp.logical_and(c == 1, s == 0))
    def _():
      pltpu.sync_copy(iids_hbm, iid_v)

      lane = lax.iota(jnp.int32, 16)

      @pl.loop(0, B // 16)
      def _(g):
        ids = iid_v[pl.ds(g * 16, 16)]
        bv = lane + g * 16
        for j in range(16):
          plsc.store_scatter(tab_v, [ids], bv, mask=lane == j)

      @pl.loop(0, B // 16)
      def _(g):
        sl = pl.ds(g * 16, 16)
        bv = lane + g * 16
        ids = iid_v[sl]
        t = plsc.load_gather(tab_v, [ids])
        iid_v[sl] = jnp.where(t == bv, ids + NU, N + (bv & 15))

      pltpu.sync_copy(iid_v, tgt_sp)

    plsc.subcore_barrier()

    @pl.when(c == 0)
    def _():
      @pl.loop(0, 2)
      def _(kk):
        b0 = s * (2 * CH) + kk * CH
        pltpu.sync_copy(uids_hbm.at[pl.ds(b0, CH)], idx_v)
        pltpu.async_copy(utab_hbm.at[idx_v], rows_v, sem).wait()
        pltpu.sync_copy(rows_v, uemb_hbm.at[pl.ds(b0, CH), :])
        pltpu.sync_copy(rows_v, xl_hbm.at[idx_v])

    @pl.when(c == 1)
    def _():
      @pl.loop(0, 2)
      def _(kk):
        b0 = s * (2 * CH) + kk * CH
        pltpu.sync_copy(iids_hbm.at[pl.ds(b0, CH)], idx_v)
        pltpu.async_copy(itab_hbm.at[idx_v], rows_v, sem).wait()
        pltpu.sync_copy(tgt_sp.at[pl.ds(b0, CH)], tgt_v)
        pltpu.sync_copy(rows_v, xl_hbm.at[tgt_v])
        pltpu.sync_copy(cemb_hbm.at[pl.ds(b0, CH), :], ce_v)
        pltpu.sync_copy(ce_v, xr_hbm.at[tgt_v])

  return k(user_ids, item_ids, c_emb, utab, itab)


def _sc_edges(src, dst, hh, ahs, ahd, inum, iden, oc, layer1):
  EC = 400
  EPT = E // 16
  NCH = EPT // EC

  mesh = plsc.VectorSubcoreMesh(**_MESH)
  den_rows = 2 * N if layer1 else N

  @functools.partial(
      pl.kernel,
      out_type=(
          jax.ShapeDtypeStruct((2 * N, oc), jnp.float32),
          jax.ShapeDtypeStruct((den_rows,), jnp.float32),
      ),
      mesh=mesh,
      compiler_params=_SC_PARAMS,
      scratch_types=(
          pltpu.VMEM((EC,), jnp.int32),
          pltpu.VMEM((EC,), jnp.int32),
          pltpu.VMEM((EC,), jnp.int32),
          pltpu.VMEM((EC,), jnp.int32),
          pltpu.VMEM((EC,), jnp.float32),
          pltpu.VMEM((EC,), jnp.float32),
          pltpu.VMEM((EC, oc), jnp.float32),
          pltpu.VMEM_SHARED((N, oc), jnp.float32),
          pltpu.VMEM_SHARED((N,), jnp.float32),
          pltpu.SemaphoreType.DMA,
      ),
  )
  def k(src_hbm, dst_hbm, hh_hbm, ahs_hbm, ahd_hbm, inum_hbm, iden_hbm,
        num_out, den_out,
        src_v, dst_v, so_v, do_v, ea_v, eb_v, rows_v, num_sp, den_sp, sem):
    c = lax.axis_index("c")
    s = lax.axis_index("s")
    coff = c * N

    def when_den(fn):
      if layer1:
        fn()
      else:
        pl.when(c == 0)(fn)

    def initcp(n0, nn):
      pltpu.sync_copy(inum_hbm.at[pl.ds(coff + n0, nn), :],
                      num_sp.at[pl.ds(n0, nn), :])

      def _den_init():
        den_src0 = (coff + n0) if layer1 else n0
        pltpu.sync_copy(iden_hbm.at[pl.ds(den_src0, nn)],
                        den_sp.at[pl.ds(n0, nn)])

      when_den(_den_init)

    @pl.when(s < 15)
    def _():
      initcp(s * 3136, 3136)

    @pl.when(s == 15)
    def _():
      initcp(15 * 3136, 2960)

    plsc.subcore_barrier()

    @pl.loop(0, NCH)
    def _(ch):
      e0 = s * EPT + ch * EC
      pltpu.sync_copy(src_hbm.at[pl.ds(e0, EC)], src_v)
      pltpu.sync_copy(dst_hbm.at[pl.ds(e0, EC)], dst_v)

      @pl.loop(0, EC // 16)
      def _(g):
        sl = pl.ds(g * 16, 16)
        so_v[sl] = src_v[sl] + coff
        if layer1:
          do_v[sl] = dst_v[sl] + coff

      if layer1:
        cp1 = pltpu.async_copy(ahs_hbm.at[so_v], ea_v, sem)
        cp2 = pltpu.async_copy(ahd_hbm.at[do_v], eb_v, sem)
      else:
        cp1 = pltpu.async_copy(ahs_hbm.at[src_v], ea_v, sem)
        cp2 = pltpu.async_copy(ahd_hbm.at[dst_v], eb_v, sem)
      cp1.wait()
      cp2.wait()

      @pl.loop(0, EC // 16)
      def _(g):
        sl = pl.ds(g * 16, 16)
        a = ea_v[sl] + eb_v[sl]
        ea_v[sl] = jnp.exp(jnp.maximum(a, 0.2 * a))

      def _den_add():
        pltpu.sync_copy(ea_v, den_sp.at[dst_v], add=True)

      when_den(_den_add)

      pltpu.async_copy(hh_hbm.at[so_v], rows_v, sem).wait()

      @pl.loop(0, EC // 16)
      def _(g):
        ev = ea_v[pl.ds(g * 16, 16)]
        for j in range(16):
          i = g * 16 + j
          sc = ev[j]
          rows_v[i, pl.ds(0, 16)] = rows_v[i, pl.ds(0, 16)] * sc
          if oc == 32:
            rows_v[i, pl.ds(16, 16)] = rows_v[i, pl.ds(16, 16)] * sc

      pltpu.sync_copy(rows_v, num_sp.at[dst_v], add=True)

    plsc.subcore_barrier()

    def outcp(n0, nn):
      pltpu.sync_copy(num_sp.at[pl.ds(n0, nn), :],
                      num_out.at[pl.ds(coff + n0, nn), :])

      def _den_out():
        den_dst0 = (coff + n0) if layer1 else n0
        pltpu.sync_copy(den_sp.at[pl.ds(n0, nn)],
                        den_out.at[pl.ds(den_dst0, nn)])

      when_den(_den_out)

    @pl.when(s < 15)
    def _():
      outcp(s * 3136, 3136)

    @pl.when(s == 15)
    def _():
      outcp(15 * 3136, 2960)

  return k(src, dst, hh, ahs, ahd, inum, iden)


def _sc_gather_out(user_ids, item_ids, x3):
  CH = 512

  mesh = plsc.VectorSubcoreMesh(**_MESH)

  @functools.partial(
      pl.kernel,
      out_type=(
          jax.ShapeDtypeStruct((B, 32), jnp.float32),
          jax.ShapeDtypeStruct((B, 32), jnp.float32),
      ),
      mesh=mesh,
      compiler_params=_SC_PARAMS,
      scratch_types=(
          pltpu.VMEM((CH,), jnp.int32),
          pltpu.VMEM((CH, 32), jnp.float32),
          pltpu.SemaphoreType.DMA,
      ),
  )
  def k(uids_hbm, iids_hbm, x3_hbm, ug_hbm, ig_hbm, idx_v, rows_v, sem):
    c = lax.axis_index("c")
    s = lax.axis_index("s")
    w = c * 16 + s
    b0 = w * CH

    pltpu.sync_copy(uids_hbm.at[pl.ds(b0, CH)], idx_v)
    pltpu.async_copy(x3_hbm.at[idx_v], rows_v, sem).wait()
    pltpu.sync_copy(rows_v, ug_hbm.at[pl.ds(b0, CH), :])

    pltpu.sync_copy(iids_hbm.at[pl.ds(b0, CH)], idx_v)

    @pl.loop(0, CH // 16)
    def _(g):
      sl = pl.ds(g * 16, 16)
      idx_v[sl] = idx_v[sl] + NU

    pltpu.async_copy(x3_hbm.at[idx_v], rows_v, sem).wait()
    pltpu.sync_copy(rows_v, ig_hbm.at[pl.ds(b0, CH), :])

  return k(user_ids, item_ids, x3)


def kernel(user_ids, item_ids, content_features, edge_index, user_emb_table,
           item_emb_table, W_content, b_content, W1, att_src1, att_dst1, b1,
           W2, att_src2, att_dst2, b2, Wp1, bp1, Wp2, bp2):
  user_ids = user_ids.astype(jnp.int32)
  item_ids = item_ids.astype(jnp.int32)
  src = edge_index[0].astype(jnp.int32)
  dst = edge_index[1].astype(jnp.int32)

  c_emb = _tc_content(content_features, W_content.T,
                      b_content.reshape(1, 32))

  xL, xR, u_emb = _sc_build_x(user_ids, item_ids, c_emb,
                              user_emb_table, item_emb_table)

  W1st = jnp.stack([W1[:, :32], W1[:, 32:]])
  hh1, ahs1, ahd1, inum1, iden1 = _tc_layer1_prep(
      xL, xR, W1st, att_src1, att_dst1)

  num1, den1 = _sc_edges(src, dst, hh1, ahs1.reshape(2 * N),
                         ahd1.reshape(2 * N), inum1, iden1.reshape(2 * N),
                         oc=32, layer1=True)

  W2st = jnp.stack([W2[:, :16], W2[:, 16:]])
  h2, a2s, a2d, inum2, iden2 = _tc_layer2_prep(
      num1, den1.reshape(2 * N, 1), b1.reshape(1, 64), W2st, W2,
      att_src2, att_dst2)

  num2, den2 = _sc_edges(src, dst, h2, a2s.reshape(N), a2d.reshape(N),
                         inum2, iden2.reshape(N), oc=16, layer1=False)

  x3 = _tc_x3(num2, den2.reshape(N, 1), b2.reshape(1, 32))

  ug, ig = _sc_gather_out(user_ids, item_ids, x3)

  out = _tc_mlp(u_emb, ig, ug, Wp1.T, bp1.reshape(1, 32), Wp2.T,
                bp2.reshape(1, 1))
  return out.reshape(B)


# speedup vs baseline: 68.8264x; 1.1217x over previous
import functools

import jax
import jax.numpy as jnp
from jax import lax
from jax.experimental import pallas as pl
from jax.experimental.pallas import tpu as pltpu
from jax.experimental.pallas import tpu_sc as plsc

N = 50000
NU = 25000
B = 16384
E = 800000
XROWS = N + 16
XP = XROWS // 4
COFF = XROWS
EPS = 1e-16

_MESH = dict(core_axis_name="c", subcore_axis_name="s", num_cores=2,
             num_subcores=16)
_SC_PARAMS = pltpu.CompilerParams(needs_layout_passes=False,
                                  use_tc_tiling_on_sc=False)

SLAB_A = 3200
SLAB_B = 2000
PH = 400
EC = 400
LANE16 = None


def _tc_content(content, WcT, bc):
  r = 2048

  def body(c_ref, w_ref, b_ref, o_ref):
    o_ref[...] = (
        jnp.dot(c_ref[...], w_ref[...], preferred_element_type=jnp.float32)
        + b_ref[...])

  return pl.pallas_call(
      body,
      grid=(B // r,),
      in_specs=[
          pl.BlockSpec((r, 128), lambda i: (i, 0)),
          pl.BlockSpec((128, 32), lambda i: (0, 0)),
          pl.BlockSpec((1, 32), lambda i: (0, 0)),
      ],
      out_specs=pl.BlockSpec((r, 32), lambda i: (i, 0)),
      out_shape=jax.ShapeDtypeStruct((B, 32), jnp.float32),
  )(content, WcT, bc)


def _tc_hmatmul(xlp, xrp, Wst):
  R = xlp.shape[0]
  H = Wst.shape[0]
  nb = 3
  r = R // nb

  def body(xl_ref, xr_ref, w_ref, o_ref):
    parts = []
    for k in range(4):
      sl = slice(32 * k, 32 * (k + 1))
      hk = (jnp.dot(xl_ref[:, sl], w_ref[0, :32, :],
                    preferred_element_type=jnp.float32)
            + jnp.dot(xr_ref[:, sl], w_ref[0, 32:, :],
                      preferred_element_type=jnp.float32))
      parts.append(hk)
    o_ref[...] = jnp.concatenate(parts, axis=1)

  return pl.pallas_call(
      body,
      grid=(H, nb),
      in_specs=[
          pl.BlockSpec((r, 128), lambda c, i: (i, 0)),
          pl.BlockSpec((r, 128), lambda c, i: (i, 0)),
          pl.BlockSpec((1, 64, 32), lambda c, i: (c, 0, 0)),
      ],
      out_specs=pl.BlockSpec((r, 128), lambda c, i: (c * nb + i, 0)),
      out_shape=jax.ShapeDtypeStruct((H * R, 128), jnp.float32),
  )(xlp, xrp, Wst)


def _tc_mlp(u_emb, igL, igR, ugL, ugR, Wp1T, bp1v, Wp2T, bp2v):
  r = 2048

  def body(ue_ref, il_ref, ir_ref, ul_ref, ur_ref, w1_ref, b1_ref, w2_ref,
           b2_ref, o_ref):
    hdn = (jnp.dot(ue_ref[...], w1_ref[0:32, :],
                   preferred_element_type=jnp.float32)
           + jnp.dot(il_ref[...], w1_ref[32:48, :],
                     preferred_element_type=jnp.float32)
           + jnp.dot(ir_ref[...], w1_ref[48:64, :],
                     preferred_element_type=jnp.float32)
           + jnp.dot(ul_ref[...], w1_ref[64:80, :],
                     preferred_element_type=jnp.float32)
           + jnp.dot(ur_ref[...], w1_ref[80:96, :],
                     preferred_element_type=jnp.float32)
           + b1_ref[...])
    hdn = jnp.maximum(hdn, 0.0)
    o_ref[...] = (jnp.dot(hdn, w2_ref[...],
                          preferred_element_type=jnp.float32) + b2_ref[...])

  return pl.pallas_call(
      body,
      grid=(B // r,),
      in_specs=[
          pl.BlockSpec((r, 32), lambda i: (i, 0)),
          pl.BlockSpec((r, 16), lambda i: (i, 0)),
          pl.BlockSpec((r, 16), lambda i: (i, 0)),
          pl.BlockSpec((r, 16), lambda i: (i, 0)),
          pl.BlockSpec((r, 16), lambda i: (i, 0)),
          pl.BlockSpec((96, 32), lambda i: (0, 0)),
          pl.BlockSpec((1, 32), lambda i: (0, 0)),
          pl.BlockSpec((32, 1), lambda i: (0, 0)),
          pl.BlockSpec((1, 1), lambda i: (0, 0)),
      ],
      out_specs=pl.BlockSpec((r, 1), lambda i: (i, 0)),
      out_shape=jax.ShapeDtypeStruct((B, 1), jnp.float32),
  )(u_emb, igL, igR, ugL, ugR, Wp1T, bp1v, Wp2T, bp2v)


def _slabs(s):
  return s * SLAB_A, SLAB_A, SLAB_B


def _sc_build_x(user_ids, item_ids, c_emb, utab, itab):
  CH = 512
  ZR = 392

  mesh = plsc.VectorSubcoreMesh(**_MESH)

  @functools.partial(
      pl.kernel,
      out_type=(
          jax.ShapeDtypeStruct((XROWS, 32), jnp.float32),
          jax.ShapeDtypeStruct((XROWS, 32), jnp.float32),
          jax.ShapeDtypeStruct((B, 32), jnp.float32),
      ),
      mesh=mesh,
      compiler_params=_SC_PARAMS,
      scratch_types=(
          pltpu.VMEM((ZR, 32), jnp.float32),
          pltpu.VMEM((CH,), jnp.int32),
          pltpu.VMEM((CH,), jnp.int32),
          pltpu.VMEM((CH, 32), jnp.float32),
          pltpu.VMEM((CH, 32), jnp.float32),
          pltpu.VMEM((B,), jnp.int32),
          pltpu.VMEM((NU,), jnp.int32),
          pltpu.VMEM_SHARED((B,), jnp.int32),
          pltpu.SemaphoreType.DMA,
      ),
  )
  def k(uids_hbm, iids_hbm, cemb_hbm, utab_hbm, itab_hbm,
        xl_hbm, xr_hbm, uemb_hbm,
        zb, idx_v, tgt_v, rows_v, ce_v, iid_v, tab_v, tgt_sp, sem):
    c = lax.axis_index("c")
    s = lax.axis_index("s")

    zero16 = jnp.zeros((16,), jnp.float32)

    @pl.loop(0, ZR)
    def _(j):
      zb[j, pl.ds(0, 16)] = zero16
      zb[j, pl.ds(16, 16)] = zero16

    row0 = c * NU + s * 1568

    def zcopy(nr, koff):
      sl = pl.ds(row0 + koff, nr)
      pltpu.sync_copy(zb.at[pl.ds(0, nr), :], xl_hbm.at[sl, :])
      pltpu.sync_copy(zb.at[pl.ds(0, nr), :], xr_hbm.at[sl, :])

    for kk in range(3):
      zcopy(ZR, kk * ZR)

    @pl.when(s < 15)
    def _():
      zcopy(ZR, 3 * ZR)

    @pl.when(jnp.logical_and(s == 15, c == 0))
    def _():
      zcopy(304, 3 * ZR)

    @pl.when(jnp.logical_and(s == 15, c == 1))
    def _():
      zcopy(320, 3 * ZR)

    @pl.when(jnp.logical_and(c == 1, s == 0))
    def _():
      pltpu.sync_copy(iids_hbm, iid_v)

      lane = lax.iota(jnp.int32, 16)

      @pl.loop(0, B // 16)
      def _(g):
        ids = iid_v[pl.ds(g * 16, 16)]
        bv = lane + g * 16
        for j in range(16):
          plsc.store_scatter(tab_v, [ids], bv, mask=lane == j)

      @pl.loop(0, B // 16)
      def _(g):
        sl = pl.ds(g * 16, 16)
        bv = lane + g * 16
        ids = iid_v[sl]
        t = plsc.load_gather(tab_v, [ids])
        iid_v[sl] = jnp.where(t == bv, ids + NU, N + (bv & 15))

      pltpu.sync_copy(iid_v, tgt_sp)

    plsc.subcore_barrier()

    @pl.when(c == 0)
    def _():
      @pl.loop(0, 2)
      def _(kk):
        b0 = s * (2 * CH) + kk * CH
        pltpu.sync_copy(uids_hbm.at[pl.ds(b0, CH)], idx_v)
        pltpu.async_copy(utab_hbm.at[idx_v], rows_v, sem).wait()
        pltpu.sync_copy(rows_v, uemb_hbm.at[pl.ds(b0, CH), :])
        pltpu.sync_copy(rows_v, xl_hbm.at[idx_v])

    @pl.when(c == 1)
    def _():
      @pl.loop(0, 2)
      def _(kk):
        b0 = s * (2 * CH) + kk * CH
        pltpu.sync_copy(iids_hbm.at[pl.ds(b0, CH)], idx_v)
        pltpu.async_copy(itab_hbm.at[idx_v], rows_v, sem).wait()
        pltpu.sync_copy(tgt_sp.at[pl.ds(b0, CH)], tgt_v)
        pltpu.sync_copy(rows_v, xl_hbm.at[tgt_v])
        pltpu.sync_copy(cemb_hbm.at[pl.ds(b0, CH), :], ce_v)
        pltpu.sync_copy(ce_v, xr_hbm.at[tgt_v])

  return k(user_ids, item_ids, c_emb, utab, itab)


def _colgather_logits(rows_v, asA, asB, adA, adB, lane):
  accs = jnp.zeros((16,), jnp.float32)
  accd = jnp.zeros((16,), jnp.float32)
  for dcol in range(32):
    cvec = jnp.full((16,), dcol, jnp.int32)
    colv = plsc.load_gather(rows_v, [lane, cvec])
    sA = asA[dcol] if dcol < 16 else asB[dcol - 16]
    sD = adA[dcol] if dcol < 16 else adB[dcol - 16]
    accs = accs + colv * sA
    accd = accd + colv * sD
  return accs, accd


def _sc_edges1(src, dst, hh, attsf, attdf, b1f):
  EPT = E // 16
  NCH = EPT // EC

  mesh = plsc.VectorSubcoreMesh(**_MESH)

  @functools.partial(
      pl.kernel,
      out_type=(
          jax.ShapeDtypeStruct((XROWS, 32), jnp.float32),
          jax.ShapeDtypeStruct((XROWS, 32), jnp.float32),
          jax.ShapeDtypeStruct((2 * COFF,), jnp.float32),
          jax.ShapeDtypeStruct((2 * COFF,), jnp.float32),
      ),
      mesh=mesh,
      compiler_params=_SC_PARAMS,
      scratch_types=(
          pltpu.VMEM((EC,), jnp.int32),
          pltpu.VMEM((EC,), jnp.int32),
          pltpu.VMEM((EC,), jnp.int32),
          pltpu.VMEM((EC,), jnp.float32),
          pltpu.VMEM((EC,), jnp.float32),
          pltpu.VMEM((EC,), jnp.float32),
          pltpu.VMEM((EC, 32), jnp.float32),
          pltpu.VMEM((64,), jnp.float32),
          pltpu.VMEM((64,), jnp.float32),
          pltpu.VMEM((64,), jnp.float32),
          pltpu.VMEM_SHARED((N, 32), jnp.float32),
          pltpu.VMEM_SHARED((N,), jnp.float32),
          pltpu.SemaphoreType.DMA,
      ),
  )
  def k(src_hbm, dst_hbm, hh_hbm, atts_hbm, attd_hbm, b1_hbm,
        x2l_hbm, x2r_hbm, ahs_hbm, ahd_hbm,
        so_v, dst_v, do_v, ea_v, eb_v, ec_v, rows_v, atts_v, attd_v, b1_v,
        num_sp, den_sp, sem):
    c = lax.axis_index("c")
    s = lax.axis_index("s")
    coff = c * COFF
    lane = lax.iota(jnp.int32, 16)

    pltpu.sync_copy(atts_hbm, atts_v)
    pltpu.sync_copy(attd_hbm, attd_v)
    pltpu.sync_copy(b1_hbm, b1_v)
    is0 = c == 0
    asA = jnp.where(is0, atts_v[pl.ds(0, 16)], atts_v[pl.ds(32, 16)])
    asB = jnp.where(is0, atts_v[pl.ds(16, 16)], atts_v[pl.ds(48, 16)])
    adA = jnp.where(is0, attd_v[pl.ds(0, 16)], attd_v[pl.ds(32, 16)])
    adB = jnp.where(is0, attd_v[pl.ds(16, 16)], attd_v[pl.ds(48, 16)])
    bA = jnp.where(is0, b1_v[pl.ds(0, 16)], b1_v[pl.ds(32, 16)])
    bB = jnp.where(is0, b1_v[pl.ds(16, 16)], b1_v[pl.ds(48, 16)])

    n0 = s * SLAB_A

    def p0_chunk(q):
      nb0 = n0 + q * PH
      pltpu.sync_copy(hh_hbm.at[pl.ds(coff + nb0, PH), :], rows_v)

      @pl.loop(0, PH // 16)
      def _(g):
        lidx = lane + g * 16
        accs, accd = _colgather_logits(rows_v, asA, asB, adA, adB, lidx)
        al = accs + accd
        ex = jnp.exp(jnp.maximum(al, 0.2 * al))
        sl = pl.ds(g * 16, 16)
        ea_v[sl] = ex
        eb_v[sl] = accs
        ec_v[sl] = accd

      pltpu.sync_copy(eb_v, ahs_hbm.at[pl.ds(coff + nb0, PH)])
      pltpu.sync_copy(ec_v, ahd_hbm.at[pl.ds(coff + nb0, PH)])
      pltpu.sync_copy(ea_v, den_sp.at[pl.ds(nb0, PH)])

      @pl.loop(0, PH // 16)
      def _(g):
        ev = ea_v[pl.ds(g * 16, 16)]
        for j in range(16):
          i = g * 16 + j
          sc = ev[j]
          rows_v[i, pl.ds(0, 16)] = rows_v[i, pl.ds(0, 16)] * sc
          rows_v[i, pl.ds(16, 16)] = rows_v[i, pl.ds(16, 16)] * sc

      pltpu.sync_copy(rows_v, num_sp.at[pl.ds(nb0, PH), :])

    @pl.when(s < 15)
    def _():
      pl.loop(0, SLAB_A // PH)(p0_chunk)

    @pl.when(s == 15)
    def _():
      pl.loop(0, SLAB_B // PH)(p0_chunk)

    plsc.subcore_barrier()

    @pl.loop(0, NCH)
    def _(ch):
      e0 = s * EPT + ch * EC
      pltpu.sync_copy(src_hbm.at[pl.ds(e0, EC)], so_v)
      pltpu.sync_copy(dst_hbm.at[pl.ds(e0, EC)], dst_v)

      @pl.loop(0, EC // 16)
      def _(g):
        sl = pl.ds(g * 16, 16)
        so_v[sl] = so_v[sl] + coff
        do_v[sl] = dst_v[sl] + coff

      cp1 = pltpu.async_copy(ahs_hbm.at[so_v], ea_v, sem)
      cp2 = pltpu.async_copy(ahd_hbm.at[do_v], eb_v, sem)
      cp1.wait()
      cp2.wait()

      @pl.loop(0, EC // 16)
      def _(g):
        sl = pl.ds(g * 16, 16)
        a = ea_v[sl] + eb_v[sl]
        ea_v[sl] = jnp.exp(jnp.maximum(a, 0.2 * a))

      pltpu.sync_copy(ea_v, den_sp.at[dst_v], add=True)
      pltpu.async_copy(hh_hbm.at[so_v], rows_v, sem).wait()

      @pl.loop(0, EC // 16)
      def _(g):
        ev = ea_v[pl.ds(g * 16, 16)]
        for j in range(16):
          i = g * 16 + j
          sc = ev[j]
          rows_v[i, pl.ds(0, 16)] = rows_v[i, pl.ds(0, 16)] * sc
          rows_v[i, pl.ds(16, 16)] = rows_v[i, pl.ds(16, 16)] * sc

      pltpu.sync_copy(rows_v, num_sp.at[dst_v], add=True)

    plsc.subcore_barrier()

    def p2_chunk(out_ref, q):
      nb0 = n0 + q * PH
      pltpu.sync_copy(num_sp.at[pl.ds(nb0, PH), :], rows_v)
      pltpu.sync_copy(den_sp.at[pl.ds(nb0, PH)], ea_v)

      @pl.loop(0, PH // 16)
      def _(g):
        sl = pl.ds(g * 16, 16)
        ea_v[sl] = 1.0 / (ea_v[sl] + EPS)

      @pl.loop(0, PH // 16)
      def _(g):
        ev = ea_v[pl.ds(g * 16, 16)]
        for j in range(16):
          i = g * 16 + j
          sc = ev[j]
          z0 = rows_v[i, pl.ds(0, 16)] * sc + bA
          z1 = rows_v[i, pl.ds(16, 16)] * sc + bB
          rows_v[i, pl.ds(0, 16)] = jnp.where(z0 > 0, z0, jnp.exp(z0) - 1.0)
          rows_v[i, pl.ds(16, 16)] = jnp.where(z1 > 0, z1, jnp.exp(z1) - 1.0)

      pltpu.sync_copy(rows_v, out_ref.at[pl.ds(nb0, PH), :])

    for cc, ref in ((0, x2l_hbm), (1, x2r_hbm)):
      @pl.when(jnp.logical_and(c == cc, s < 15))
      def _(ref=ref):
        pl.loop(0, SLAB_A // PH)(functools.partial(p2_chunk, ref))

      @pl.when(jnp.logical_and(c == cc, s == 15))
      def _(ref=ref):
        pl.loop(0, SLAB_B // PH)(functools.partial(p2_chunk, ref))

  return k(src, dst, hh, attsf, attdf, b1f)


def _sc_edges2(src, dst, h2, atts2, attd2, b2f):
  EPT = E // 16
  NCH = EPT // EC

  mesh = plsc.VectorSubcoreMesh(**_MESH)

  @functools.partial(
      pl.kernel,
      out_type=(
          jax.ShapeDtypeStruct((N, 16), jnp.float32),
          jax.ShapeDtypeStruct((N, 16), jnp.float32),
          jax.ShapeDtypeStruct((N,), jnp.float32),
          jax.ShapeDtypeStruct((N,), jnp.float32),
      ),
      mesh=mesh,
      compiler_params=_SC_PARAMS,
      scratch_types=(
          pltpu.VMEM((EC,), jnp.int32),
          pltpu.VMEM((EC,), jnp.int32),
          pltpu.VMEM((EC,), jnp.float32),
          pltpu.VMEM((EC,), jnp.float32),
          pltpu.VMEM((EC,), jnp.float32),
          pltpu.VMEM((EC, 32), jnp.float32),
          pltpu.VMEM((EC, 16), jnp.float32),
          pltpu.VMEM((32,), jnp.float32),
          pltpu.VMEM((32,), jnp.float32),
          pltpu.VMEM((32,), jnp.float32),
          pltpu.VMEM_SHARED((N, 16), jnp.float32),
          pltpu.VMEM_SHARED((N,), jnp.float32),
          pltpu.SemaphoreType.DMA,
      ),
  )
  def k(src_hbm, dst_hbm, h2_hbm, atts_hbm, attd_hbm, b2_hbm,
        x3a_hbm, x3b_hbm, ahs_hbm, ahd_hbm,
        so_v, dst_v, ea_v, eb_v, ec_v, rows_v, half_v, atts_v, attd_v, b2_v,
        num_sp, den_sp, sem):
    c = lax.axis_index("c")
    s = lax.axis_index("s")
    lane = lax.iota(jnp.int32, 16)

    pltpu.sync_copy(atts_hbm, atts_v)
    pltpu.sync_copy(attd_hbm, attd_v)
    pltpu.sync_copy(b2_hbm, b2_v)
    is0 = c == 0
    asA = atts_v[pl.ds(0, 16)]
    asB = atts_v[pl.ds(16, 16)]
    adA = attd_v[pl.ds(0, 16)]
    adB = attd_v[pl.ds(16, 16)]
    bH = jnp.where(is0, b2_v[pl.ds(0, 16)], b2_v[pl.ds(16, 16)])

    n0 = s * SLAB_A

    def half_row(i):
      r0 = rows_v[i, pl.ds(0, 16)]
      r1 = rows_v[i, pl.ds(16, 16)]
      return jnp.where(is0, r0, r1)

    def p0_chunk(q):
      nb0 = n0 + q * PH
      pltpu.sync_copy(h2_hbm.at[pl.ds(nb0, PH), :], rows_v)

      @pl.loop(0, PH // 16)
      def _(g):
        lidx = lane + g * 16
        accs, accd = _colgather_logits(rows_v, asA, asB, adA, adB, lidx)
        al = accs + accd
        ex = jnp.exp(jnp.maximum(al, 0.2 * al))
        sl = pl.ds(g * 16, 16)
        ea_v[sl] = ex
        eb_v[sl] = accs
        ec_v[sl] = accd

      pltpu.sync_copy(eb_v, ahs_hbm.at[pl.ds(nb0, PH)])
      pltpu.sync_copy(ec_v, ahd_hbm.at[pl.ds(nb0, PH)])
      pltpu.sync_copy(ea_v, den_sp.at[pl.ds(nb0, PH)])

      @pl.loop(0, PH // 16)
      def _(g):
        ev = ea_v[pl.ds(g * 16, 16)]
        for j in range(16):
          i = g * 16 + j
          half_v[i, pl.ds(0, 16)] = half_row(i) * ev[j]

      pltpu.sync_copy(half_v, num_sp.at[pl.ds(nb0, PH), :])

    @pl.when(s < 15)
    def _():
      pl.loop(0, SLAB_A // PH)(p0_chunk)

    @pl.when(s == 15)
    def _():
      pl.loop(0, SLAB_B // PH)(p0_chunk)

    plsc.subcore_barrier()

    @pl.loop(0, NCH)
    def _(ch):
      e0 = s * EPT + ch * EC
      pltpu.sync_copy(src_hbm.at[pl.ds(e0, EC)], so_v)
      pltpu.sync_copy(dst_hbm.at[pl.ds(e0, EC)], dst_v)

      cp1 = pltpu.async_copy(ahs_hbm.at[so_v], ea_v, sem)
      cp2 = pltpu.async_copy(ahd_hbm.at[dst_v], eb_v, sem)
      cp1.wait()
      cp2.wait()

      @pl.loop(0, EC // 16)
      def _(g):
        sl = pl.ds(g * 16, 16)
        a = ea_v[sl] + eb_v[sl]
        ea_v[sl] = jnp.exp(jnp.maximum(a, 0.2 * a))

      pltpu.sync_copy(ea_v, den_sp.at[dst_v], add=True)
      pltpu.async_copy(h2_hbm.at[so_v], rows_v, sem).wait()

      @pl.loop(0, EC // 16)
      def _(g):
        ev = ea_v[pl.ds(g * 16, 16)]
        for j in range(16):
          i = g * 16 + j
          half_v[i, pl.ds(0, 16)] = half_row(i) * ev[j]

      pltpu.sync_copy(half_v, num_sp.at[dst_v], add=True)

    plsc.subcore_barrier()

    def p2_chunk(out_ref, q):
      nb0 = n0 + q * PH
      pltpu.sync_copy(num_sp.at[pl.ds(nb0, PH), :], half_v)
      pltpu.sync_copy(den_sp.at[pl.ds(nb0, PH)], ea_v)

      @pl.loop(0, PH // 16)
      def _(g):
        sl = pl.ds(g * 16, 16)
        ea_v[sl] = 1.0 / (ea_v[sl] + EPS)

      @pl.loop(0, PH // 16)
      def _(g):
        ev = ea_v[pl.ds(g * 16, 16)]
        for j in range(16):
          i = g * 16 + j
          half_v[i, pl.ds(0, 16)] = half_v[i, pl.ds(0, 16)] * ev[j] + bH

      pltpu.sync_copy(half_v, out_ref.at[pl.ds(nb0, PH), :])

    for cc, ref in ((0, x3a_hbm), (1, x3b_hbm)):
      @pl.when(jnp.logical_and(c == cc, s < 15))
      def _(ref=ref):
        pl.loop(0, SLAB_A // PH)(functools.partial(p2_chunk, ref))

      @pl.when(jnp.logical_and(c == cc, s == 15))
      def _(ref=ref):
        pl.loop(0, SLAB_B // PH)(functools.partial(p2_chunk, ref))

  return k(src, dst, h2, atts2, attd2, b2f)


def _sc_gather_out(user_ids, item_ids, x3a, x3b):
  CH = 512

  mesh = plsc.VectorSubcoreMesh(**_MESH)

  @functools.partial(
      pl.kernel,
      out_type=tuple(
          jax.ShapeDtypeStruct((B, 16), jnp.float32) for _ in range(4)),
      mesh=mesh,
      compiler_params=_SC_PARAMS,
      scratch_types=(
          pltpu.VMEM((CH,), jnp.int32),
          pltpu.VMEM((CH, 16), jnp.float32),
          pltpu.SemaphoreType.DMA,
      ),
  )
  def k(uids_hbm, iids_hbm, x3a_hbm, x3b_hbm,
        ugl_hbm, ugr_hbm, igl_hbm, igr_hbm, idx_v, rows_v, sem):
    c = lax.axis_index("c")
    s = lax.axis_index("s")
    w = c * 16 + s
    b0 = w * CH

    pltpu.sync_copy(uids_hbm.at[pl.ds(b0, CH)], idx_v)
    pltpu.async_copy(x3a_hbm.at[idx_v], rows_v, sem).wait()
    pltpu.sync_copy(rows_v, ugl_hbm.at[pl.ds(b0, CH), :])
    pltpu.async_copy(x3b_hbm.at[idx_v], rows_v, sem).wait()
    pltpu.sync_copy(rows_v, ugr_hbm.at[pl.ds(b0, CH), :])

    pltpu.sync_copy(iids_hbm.at[pl.ds(b0, CH)], idx_v)

    @pl.loop(0, CH // 16)
    def _(g):
      sl = pl.ds(g * 16, 16)
      idx_v[sl] = idx_v[sl] + NU

    pltpu.async_copy(x3a_hbm.at[idx_v], rows_v, sem).wait()
    pltpu.sync_copy(rows_v, igl_hbm.at[pl.ds(b0, CH), :])
    pltpu.async_copy(x3b_hbm.at[idx_v], rows_v, sem).wait()
    pltpu.sync_copy(rows_v, igr_hbm.at[pl.ds(b0, CH), :])

  return k(user_ids, item_ids, x3a, x3b)


def kernel(user_ids, item_ids, content_features, edge_index, user_emb_table,
           item_emb_table, W_content, b_content, W1, att_src1, att_dst1, b1,
           W2, att_src2, att_dst2, b2, Wp1, bp1, Wp2, bp2):
  user_ids = user_ids.astype(jnp.int32)
  item_ids = item_ids.astype(jnp.int32)
  src = edge_index[0].astype(jnp.int32)
  dst = edge_index[1].astype(jnp.int32)

  c_emb = _tc_content(content_features, W_content.T,
                      b_content.reshape(1, 32))

  xL, xR, u_emb = _sc_build_x(user_ids, item_ids, c_emb,
                              user_emb_table, item_emb_table)

  W1st = jnp.stack([W1[:, :32], W1[:, 32:]])
  hh1p = _tc_hmatmul(xL.reshape(XP, 128), xR.reshape(XP, 128), W1st)
  hh1 = hh1p.reshape(2 * XROWS, 32)

  x2L, x2R, _, _ = _sc_edges1(src, dst, hh1, att_src1.reshape(64),
                              att_dst1.reshape(64), b1)

  h2p = _tc_hmatmul(x2L.reshape(XP, 128), x2R.reshape(XP, 128),
                    W2.reshape(1, 64, 32))
  h2 = h2p.reshape(XROWS, 32)

  x3a, x3b, _, _ = _sc_edges2(src, dst, h2, att_src2.reshape(32),
                              att_dst2.reshape(32), b2)

  ugL, ugR, igL, igR = _sc_gather_out(user_ids, item_ids, x3a, x3b)

  out = _tc_mlp(u_emb, igL, igR, ugL, ugR, Wp1.T, bp1.reshape(1, 32),
                Wp2.T, bp2.reshape(1, 1))
  return out.reshape(B)


# speedup vs baseline: 102.5657x; 1.4902x over previous
import functools

import jax
import jax.numpy as jnp
from jax import lax
from jax.experimental import pallas as pl
from jax.experimental.pallas import tpu as pltpu
from jax.experimental.pallas import tpu_sc as plsc

N = 50000
NU = 25000
B = 16384
E = 800000
XROWS = N + 16
XP = XROWS // 4
COFF = XROWS
EPS = 1e-16

_MESH = dict(core_axis_name="c", subcore_axis_name="s", num_cores=2,
             num_subcores=16)
_SC_PARAMS = pltpu.CompilerParams(needs_layout_passes=False,
                                  use_tc_tiling_on_sc=False)

SLAB_A = 3200
SLAB_B = 2000
PH = 400
EC = 400
LANE16 = None


def _tc_content(content, WcT, bc):
  r = 2048

  def body(c_ref, w_ref, b_ref, o_ref):
    o_ref[...] = (
        jnp.dot(c_ref[...], w_ref[...], preferred_element_type=jnp.float32)
        + b_ref[...])

  return pl.pallas_call(
      body,
      grid=(B // r,),
      in_specs=[
          pl.BlockSpec((r, 128), lambda i: (i, 0)),
          pl.BlockSpec((128, 32), lambda i: (0, 0)),
          pl.BlockSpec((1, 32), lambda i: (0, 0)),
      ],
      out_specs=pl.BlockSpec((r, 32), lambda i: (i, 0)),
      out_shape=jax.ShapeDtypeStruct((B, 32), jnp.float32),
  )(content, WcT, bc)


def _tc_hmatmul(xlp, xrp, Wst):
  R = xlp.shape[0]
  H = Wst.shape[0]
  nb = 3
  r = R // nb

  def body(xl_ref, xr_ref, w_ref, o_ref):
    parts = []
    for k in range(4):
      sl = slice(32 * k, 32 * (k + 1))
      hk = (jnp.dot(xl_ref[:, sl], w_ref[0, :32, :],
                    preferred_element_type=jnp.float32)
            + jnp.dot(xr_ref[:, sl], w_ref[0, 32:, :],
                      preferred_element_type=jnp.float32))
      parts.append(hk)
    o_ref[...] = jnp.concatenate(parts, axis=1)

  return pl.pallas_call(
      body,
      grid=(H, nb),
      in_specs=[
          pl.BlockSpec((r, 128), lambda c, i: (i, 0)),
          pl.BlockSpec((r, 128), lambda c, i: (i, 0)),
          pl.BlockSpec((1, 64, 32), lambda c, i: (c, 0, 0)),
      ],
      out_specs=pl.BlockSpec((r, 128), lambda c, i: (c * nb + i, 0)),
      out_shape=jax.ShapeDtypeStruct((H * R, 128), jnp.float32),
  )(xlp, xrp, Wst)


def _tc_mlp(u_emb, igL, igR, ugL, ugR, Wp1T, bp1v, Wp2T, bp2v):
  r = 2048

  def body(ue_ref, il_ref, ir_ref, ul_ref, ur_ref, w1_ref, b1_ref, w2_ref,
           b2_ref, o_ref):
    hdn = (jnp.dot(ue_ref[...], w1_ref[0:32, :],
                   preferred_element_type=jnp.float32)
           + jnp.dot(il_ref[...], w1_ref[32:48, :],
                     preferred_element_type=jnp.float32)
           + jnp.dot(ir_ref[...], w1_ref[48:64, :],
                     preferred_element_type=jnp.float32)
           + jnp.dot(ul_ref[...], w1_ref[64:80, :],
                     preferred_element_type=jnp.float32)
           + jnp.dot(ur_ref[...], w1_ref[80:96, :],
                     preferred_element_type=jnp.float32)
           + b1_ref[...])
    hdn = jnp.maximum(hdn, 0.0)
    o_ref[...] = (jnp.dot(hdn, w2_ref[...],
                          preferred_element_type=jnp.float32) + b2_ref[...])

  return pl.pallas_call(
      body,
      grid=(B // r,),
      in_specs=[
          pl.BlockSpec((r, 32), lambda i: (i, 0)),
          pl.BlockSpec((r, 16), lambda i: (i, 0)),
          pl.BlockSpec((r, 16), lambda i: (i, 0)),
          pl.BlockSpec((r, 16), lambda i: (i, 0)),
          pl.BlockSpec((r, 16), lambda i: (i, 0)),
          pl.BlockSpec((96, 32), lambda i: (0, 0)),
          pl.BlockSpec((1, 32), lambda i: (0, 0)),
          pl.BlockSpec((32, 1), lambda i: (0, 0)),
          pl.BlockSpec((1, 1), lambda i: (0, 0)),
      ],
      out_specs=pl.BlockSpec((r, 1), lambda i: (i, 0)),
      out_shape=jax.ShapeDtypeStruct((B, 1), jnp.float32),
  )(u_emb, igL, igR, ugL, ugR, Wp1T, bp1v, Wp2T, bp2v)


def _slabs(s):
  return s * SLAB_A, SLAB_A, SLAB_B


def _sc_build_x(user_ids, item_ids, c_emb, utab, itab):
  CH = 512
  ZR = 392

  mesh = plsc.VectorSubcoreMesh(**_MESH)

  @functools.partial(
      pl.kernel,
      out_type=(
          jax.ShapeDtypeStruct((XROWS, 32), jnp.float32),
          jax.ShapeDtypeStruct((XROWS, 32), jnp.float32),
          jax.ShapeDtypeStruct((B, 32), jnp.float32),
      ),
      mesh=mesh,
      compiler_params=_SC_PARAMS,
      scratch_types=(
          pltpu.VMEM((ZR, 32), jnp.float32),
          pltpu.VMEM((CH,), jnp.int32),
          pltpu.VMEM((CH,), jnp.int32),
          pltpu.VMEM((CH, 32), jnp.float32),
          pltpu.VMEM((CH, 32), jnp.float32),
          pltpu.VMEM((B,), jnp.int32),
          pltpu.VMEM((NU,), jnp.int32),
          pltpu.VMEM_SHARED((B,), jnp.int32),
          pltpu.SemaphoreType.DMA,
      ),
  )
  def k(uids_hbm, iids_hbm, cemb_hbm, utab_hbm, itab_hbm,
        xl_hbm, xr_hbm, uemb_hbm,
        zb, idx_v, tgt_v, rows_v, ce_v, iid_v, tab_v, tgt_sp, sem):
    c = lax.axis_index("c")
    s = lax.axis_index("s")

    zero16 = jnp.zeros((16,), jnp.float32)

    @pl.loop(0, ZR)
    def _(j):
      zb[j, pl.ds(0, 16)] = zero16
      zb[j, pl.ds(16, 16)] = zero16

    row0 = c * NU + s * 1568

    def zcopy(nr, koff):
      sl = pl.ds(row0 + koff, nr)
      pltpu.sync_copy(zb.at[pl.ds(0, nr), :], xl_hbm.at[sl, :])
      pltpu.sync_copy(zb.at[pl.ds(0, nr), :], xr_hbm.at[sl, :])

    for kk in range(3):
      zcopy(ZR, kk * ZR)

    @pl.when(s < 15)
    def _():
      zcopy(ZR, 3 * ZR)

    @pl.when(jnp.logical_and(s == 15, c == 0))
    def _():
      zcopy(304, 3 * ZR)

    @pl.when(jnp.logical_and(s == 15, c == 1))
    def _():
      zcopy(320, 3 * ZR)

    @pl.when(jnp.logical_and(c == 1, s == 0))
    def _():
      pltpu.sync_copy(iids_hbm, iid_v)

      lane = lax.iota(jnp.int32, 16)

      @pl.loop(0, B // 16)
      def _(g):
        ids = iid_v[pl.ds(g * 16, 16)]
        bv = lane + g * 16
        for j in range(16):
          plsc.store_scatter(tab_v, [ids], bv, mask=lane == j)

      @pl.loop(0, B // 16)
      def _(g):
        sl = pl.ds(g * 16, 16)
        bv = lane + g * 16
        ids = iid_v[sl]
        t = plsc.load_gather(tab_v, [ids])
        iid_v[sl] = jnp.where(t == bv, ids + NU, N + (bv & 15))

      pltpu.sync_copy(iid_v, tgt_sp)

    plsc.subcore_barrier()

    @pl.when(c == 0)
    def _():
      @pl.loop(0, 2)
      def _(kk):
        b0 = s * (2 * CH) + kk * CH
        pltpu.sync_copy(uids_hbm.at[pl.ds(b0, CH)], idx_v)
        pltpu.async_copy(utab_hbm.at[idx_v], rows_v, sem).wait()
        pltpu.sync_copy(rows_v, uemb_hbm.at[pl.ds(b0, CH), :])
        pltpu.sync_copy(rows_v, xl_hbm.at[idx_v])

    @pl.when(c == 1)
    def _():
      @pl.loop(0, 2)
      def _(kk):
        b0 = s * (2 * CH) + kk * CH
        pltpu.sync_copy(iids_hbm.at[pl.ds(b0, CH)], idx_v)
        pltpu.async_copy(itab_hbm.at[idx_v], rows_v, sem).wait()
        pltpu.sync_copy(tgt_sp.at[pl.ds(b0, CH)], tgt_v)
        pltpu.sync_copy(rows_v, xl_hbm.at[tgt_v])
        pltpu.sync_copy(cemb_hbm.at[pl.ds(b0, CH), :], ce_v)
        pltpu.sync_copy(ce_v, xr_hbm.at[tgt_v])

  return k(user_ids, item_ids, c_emb, utab, itab)


def _colgather_logits(rows_v, asA, asB, adA, adB, lane):
  accs = jnp.zeros((16,), jnp.float32)
  accd = jnp.zeros((16,), jnp.float32)
  for dcol in range(32):
    cvec = jnp.full((16,), dcol, jnp.int32)
    colv = plsc.load_gather(rows_v, [lane, cvec])
    sA = asA[dcol] if dcol < 16 else asB[dcol - 16]
    sD = adA[dcol] if dcol < 16 else adB[dcol - 16]
    accs = accs + colv * sA
    accd = accd + colv * sD
  return accs, accd


def _sc_edges1(src, dst, hh, attsf, attdf, b1f):
  EPT = E // 16
  NCH = EPT // EC

  mesh = plsc.VectorSubcoreMesh(**_MESH)

  @functools.partial(
      pl.kernel,
      out_type=(
          jax.ShapeDtypeStruct((XROWS, 32), jnp.float32),
          jax.ShapeDtypeStruct((XROWS, 32), jnp.float32),
          jax.ShapeDtypeStruct((2 * COFF,), jnp.float32),
          jax.ShapeDtypeStruct((2 * COFF,), jnp.float32),
      ),
      mesh=mesh,
      compiler_params=_SC_PARAMS,
      scratch_types=(
          pltpu.VMEM((EC,), jnp.int32),
          pltpu.VMEM((EC,), jnp.int32),
          pltpu.VMEM((EC,), jnp.int32),
          pltpu.VMEM((EC,), jnp.float32),
          pltpu.VMEM((EC,), jnp.float32),
          pltpu.VMEM((EC,), jnp.float32),
          pltpu.VMEM((EC,), jnp.int32),
          pltpu.VMEM((EC,), jnp.int32),
          pltpu.VMEM((EC,), jnp.float32),
          pltpu.VMEM((EC,), jnp.float32),
          pltpu.VMEM((EC, 32), jnp.float32),
          pltpu.VMEM((64,), jnp.float32),
          pltpu.VMEM((64,), jnp.float32),
          pltpu.VMEM((64,), jnp.float32),
          pltpu.VMEM_SHARED((N, 32), jnp.float32),
          pltpu.VMEM_SHARED((N,), jnp.float32),
          pltpu.SemaphoreType.DMA,
          pltpu.SemaphoreType.DMA,
          pltpu.SemaphoreType.DMA,
      ),
  )
  def k(src_hbm, dst_hbm, hh_hbm, atts_hbm, attd_hbm, b1_hbm,
        x2l_hbm, x2r_hbm, ahs_hbm, ahd_hbm,
        so_v, dst_v, do_v, ea_v, eb_v, ec_v, so2_v, do2_v, ea2_v, eb2_v,
        rows_v, atts_v, attd_v, b1_v,
        num_sp, den_sp, sem, sem2, sem3):
    c = lax.axis_index("c")
    s = lax.axis_index("s")
    coff = c * COFF
    lane = lax.iota(jnp.int32, 16)

    pltpu.sync_copy(atts_hbm, atts_v)
    pltpu.sync_copy(attd_hbm, attd_v)
    pltpu.sync_copy(b1_hbm, b1_v)
    is0 = c == 0
    asA = jnp.where(is0, atts_v[pl.ds(0, 16)], atts_v[pl.ds(32, 16)])
    asB = jnp.where(is0, atts_v[pl.ds(16, 16)], atts_v[pl.ds(48, 16)])
    adA = jnp.where(is0, attd_v[pl.ds(0, 16)], attd_v[pl.ds(32, 16)])
    adB = jnp.where(is0, attd_v[pl.ds(16, 16)], attd_v[pl.ds(48, 16)])
    bA = jnp.where(is0, b1_v[pl.ds(0, 16)], b1_v[pl.ds(32, 16)])
    bB = jnp.where(is0, b1_v[pl.ds(16, 16)], b1_v[pl.ds(48, 16)])

    n0 = s * SLAB_A

    def p0_chunk(q):
      nb0 = n0 + q * PH
      pltpu.sync_copy(hh_hbm.at[pl.ds(coff + nb0, PH), :], rows_v)

      @pl.loop(0, PH // 16)
      def _(g):
        lidx = lane + g * 16
        accs, accd = _colgather_logits(rows_v, asA, asB, adA, adB, lidx)
        al = accs + accd
        ex = jnp.exp(jnp.maximum(al, 0.2 * al))
        sl = pl.ds(g * 16, 16)
        ea_v[sl] = ex
        eb_v[sl] = accs
        ec_v[sl] = accd

      pltpu.sync_copy(eb_v, ahs_hbm.at[pl.ds(coff + nb0, PH)])
      pltpu.sync_copy(ec_v, ahd_hbm.at[pl.ds(coff + nb0, PH)])
      pltpu.sync_copy(ea_v, den_sp.at[pl.ds(nb0, PH)])

      @pl.loop(0, PH // 16)
      def _(g):
        ev = ea_v[pl.ds(g * 16, 16)]
        for j in range(16):
          i = g * 16 + j
          sc = ev[j]
          rows_v[i, pl.ds(0, 16)] = rows_v[i, pl.ds(0, 16)] * sc
          rows_v[i, pl.ds(16, 16)] = rows_v[i, pl.ds(16, 16)] * sc

      pltpu.sync_copy(rows_v, num_sp.at[pl.ds(nb0, PH), :])

    @pl.when(s < 15)
    def _():
      pl.loop(0, SLAB_A // PH)(p0_chunk)

    @pl.when(s == 15)
    def _():
      pl.loop(0, SLAB_B // PH)(p0_chunk)

    plsc.subcore_barrier()

    def issue(so, do, ea, eb, gsem, ch):
      e0 = s * EPT + ch * EC
      pltpu.sync_copy(src_hbm.at[pl.ds(e0, EC)], so)
      pltpu.sync_copy(dst_hbm.at[pl.ds(e0, EC)], do)

      @pl.loop(0, EC // 16)
      def _(g):
        sl = pl.ds(g * 16, 16)
        so[sl] = so[sl] + coff
        do[sl] = do[sl] + coff

      pltpu.async_copy(ahs_hbm.at[so], ea, gsem)
      pltpu.async_copy(ahd_hbm.at[do], eb, gsem)

    def process(so, do, ea, eb, gsem, ch, prefetch):
      pltpu.async_copy(hh_hbm.at[so], rows_v, sem2)
      if prefetch is not None:
        issue(*prefetch, ch + 1)
      pltpu.make_async_copy(ahs_hbm.at[so], ea, gsem).wait()
      pltpu.make_async_copy(ahd_hbm.at[do], eb, gsem).wait()

      @pl.loop(0, EC // 16)
      def _(g):
        sl = pl.ds(g * 16, 16)
        a = ea[sl] + eb[sl]
        ea[sl] = jnp.exp(jnp.maximum(a, 0.2 * a))
        dst_v[sl] = do[sl] - coff

      pltpu.sync_copy(ea, den_sp.at[dst_v], add=True)
      pltpu.make_async_copy(hh_hbm.at[so], rows_v, sem2).wait()

      @pl.loop(0, EC // 16)
      def _(g):
        ev = ea[pl.ds(g * 16, 16)]
        for j in range(16):
          i = g * 16 + j
          sc = ev[j]
          rows_v[i, pl.ds(0, 16)] = rows_v[i, pl.ds(0, 16)] * sc
          rows_v[i, pl.ds(16, 16)] = rows_v[i, pl.ds(16, 16)] * sc

      pltpu.sync_copy(rows_v, num_sp.at[dst_v], add=True)

    bufA = (so_v, do_v, ea_v, eb_v, sem)
    bufB = (so2_v, do2_v, ea2_v, eb2_v, sem3)
    issue(*bufA, 0)

    @pl.loop(0, (NCH - 1) // 2)
    def _(g):
      process(*bufA, 2 * g, prefetch=bufB)
      process(*bufB, 2 * g + 1, prefetch=bufA)

    process(*bufA, NCH - 1, prefetch=None)

    plsc.subcore_barrier()

    def p2_chunk(out_ref, q):
      nb0 = n0 + q * PH
      pltpu.sync_copy(num_sp.at[pl.ds(nb0, PH), :], rows_v)
      pltpu.sync_copy(den_sp.at[pl.ds(nb0, PH)], ea_v)

      @pl.loop(0, PH // 16)
      def _(g):
        sl = pl.ds(g * 16, 16)
        ea_v[sl] = 1.0 / (ea_v[sl] + EPS)

      @pl.loop(0, PH // 16)
      def _(g):
        ev = ea_v[pl.ds(g * 16, 16)]
        for j in range(16):
          i = g * 16 + j
          sc = ev[j]
          z0 = rows_v[i, pl.ds(0, 16)] * sc + bA
          z1 = rows_v[i, pl.ds(16, 16)] * sc + bB
          rows_v[i, pl.ds(0, 16)] = jnp.where(z0 > 0, z0, jnp.exp(z0) - 1.0)
          rows_v[i, pl.ds(16, 16)] = jnp.where(z1 > 0, z1, jnp.exp(z1) - 1.0)

      pltpu.sync_copy(rows_v, out_ref.at[pl.ds(nb0, PH), :])

    for cc, ref in ((0, x2l_hbm), (1, x2r_hbm)):
      @pl.when(jnp.logical_and(c == cc, s < 15))
      def _(ref=ref):
        pl.loop(0, SLAB_A // PH)(functools.partial(p2_chunk, ref))

      @pl.when(jnp.logical_and(c == cc, s == 15))
      def _(ref=ref):
        pl.loop(0, SLAB_B // PH)(functools.partial(p2_chunk, ref))

  return k(src, dst, hh, attsf, attdf, b1f)


def _sc_edges2(src, dst, h2, atts2, attd2, b2f):
  EPT = E // 16
  NCH = EPT // EC

  mesh = plsc.VectorSubcoreMesh(**_MESH)

  @functools.partial(
      pl.kernel,
      out_type=(
          jax.ShapeDtypeStruct((N, 16), jnp.float32),
          jax.ShapeDtypeStruct((N, 16), jnp.float32),
          jax.ShapeDtypeStruct((N,), jnp.float32),
          jax.ShapeDtypeStruct((N,), jnp.float32),
      ),
      mesh=mesh,
      compiler_params=_SC_PARAMS,
      scratch_types=(
          pltpu.VMEM((EC,), jnp.int32),
          pltpu.VMEM((EC,), jnp.int32),
          pltpu.VMEM((EC,), jnp.float32),
          pltpu.VMEM((EC,), jnp.float32),
          pltpu.VMEM((EC,), jnp.float32),
          pltpu.VMEM((EC,), jnp.int32),
          pltpu.VMEM((EC,), jnp.int32),
          pltpu.VMEM((EC,), jnp.float32),
          pltpu.VMEM((EC,), jnp.float32),
          pltpu.VMEM((EC, 32), jnp.float32),
          pltpu.VMEM((EC, 16), jnp.float32),
          pltpu.VMEM((32,), jnp.float32),
          pltpu.VMEM((32,), jnp.float32),
          pltpu.VMEM((32,), jnp.float32),
          pltpu.VMEM_SHARED((N, 16), jnp.float32),
          pltpu.VMEM_SHARED((N,), jnp.float32),
          pltpu.SemaphoreType.DMA,
          pltpu.SemaphoreType.DMA,
          pltpu.SemaphoreType.DMA,
      ),
  )
  def k(src_hbm, dst_hbm, h2_hbm, atts_hbm, attd_hbm, b2_hbm,
        x3a_hbm, x3b_hbm, ahs_hbm, ahd_hbm,
        so_v, dst_v, ea_v, eb_v, ec_v, so2_v, dst2_v, ea2_v, eb2_v,
        rows_v, half_v, atts_v, attd_v, b2_v,
        num_sp, den_sp, sem, sem2, sem3):
    c = lax.axis_index("c")
    s = lax.axis_index("s")
    lane = lax.iota(jnp.int32, 16)

    pltpu.sync_copy(atts_hbm, atts_v)
    pltpu.sync_copy(attd_hbm, attd_v)
    pltpu.sync_copy(b2_hbm, b2_v)
    is0 = c == 0
    asA = atts_v[pl.ds(0, 16)]
    asB = atts_v[pl.ds(16, 16)]
    adA = attd_v[pl.ds(0, 16)]
    adB = attd_v[pl.ds(16, 16)]
    bH = jnp.where(is0, b2_v[pl.ds(0, 16)], b2_v[pl.ds(16, 16)])

    n0 = s * SLAB_A

    def half_row(i):
      r0 = rows_v[i, pl.ds(0, 16)]
      r1 = rows_v[i, pl.ds(16, 16)]
      return jnp.where(is0, r0, r1)

    def p0_chunk(q):
      nb0 = n0 + q * PH
      pltpu.sync_copy(h2_hbm.at[pl.ds(nb0, PH), :], rows_v)

      @pl.loop(0, PH // 16)
      def _(g):
        lidx = lane + g * 16
        accs, accd = _colgather_logits(rows_v, asA, asB, adA, adB, lidx)
        al = accs + accd
        ex = jnp.exp(jnp.maximum(al, 0.2 * al))
        sl = pl.ds(g * 16, 16)
        ea_v[sl] = ex
        eb_v[sl] = accs
        ec_v[sl] = accd

      pltpu.sync_copy(eb_v, ahs_hbm.at[pl.ds(nb0, PH)])
      pltpu.sync_copy(ec_v, ahd_hbm.at[pl.ds(nb0, PH)])
      pltpu.sync_copy(ea_v, den_sp.at[pl.ds(nb0, PH)])

      @pl.loop(0, PH // 16)
      def _(g):
        ev = ea_v[pl.ds(g * 16, 16)]
        for j in range(16):
          i = g * 16 + j
          half_v[i, pl.ds(0, 16)] = half_row(i) * ev[j]

      pltpu.sync_copy(half_v, num_sp.at[pl.ds(nb0, PH), :])

    @pl.when(s < 15)
    def _():
      pl.loop(0, SLAB_A // PH)(p0_chunk)

    @pl.when(s == 15)
    def _():
      pl.loop(0, SLAB_B // PH)(p0_chunk)

    plsc.subcore_barrier()

    def issue(so, dv, ea, eb, gsem, ch):
      e0 = s * EPT + ch * EC
      pltpu.sync_copy(src_hbm.at[pl.ds(e0, EC)], so)
      pltpu.sync_copy(dst_hbm.at[pl.ds(e0, EC)], dv)
      pltpu.async_copy(ahs_hbm.at[so], ea, gsem)
      pltpu.async_copy(ahd_hbm.at[dv], eb, gsem)

    def process(so, dv, ea, eb, gsem, ch, prefetch):
      pltpu.async_copy(h2_hbm.at[so], rows_v, sem2)
      if prefetch is not None:
        issue(*prefetch, ch + 1)
      pltpu.make_async_copy(ahs_hbm.at[so], ea, gsem).wait()
      pltpu.make_async_copy(ahd_hbm.at[dv], eb, gsem).wait()

      @pl.loop(0, EC // 16)
      def _(g):
        sl = pl.ds(g * 16, 16)
        a = ea[sl] + eb[sl]
        ea[sl] = jnp.exp(jnp.maximum(a, 0.2 * a))

      pltpu.sync_copy(ea, den_sp.at[dv], add=True)
      pltpu.make_async_copy(h2_hbm.at[so], rows_v, sem2).wait()

      @pl.loop(0, EC // 16)
      def _(g):
        ev = ea[pl.ds(g * 16, 16)]
        for j in range(16):
          i = g * 16 + j
          half_v[i, pl.ds(0, 16)] = half_row(i) * ev[j]

      pltpu.sync_copy(half_v, num_sp.at[dv], add=True)

    bufA = (so_v, dst_v, ea_v, eb_v, sem)
    bufB = (so2_v, dst2_v, ea2_v, eb2_v, sem3)
    issue(*bufA, 0)

    @pl.loop(0, (NCH - 1) // 2)
    def _(g):
      process(*bufA, 2 * g, prefetch=bufB)
      process(*bufB, 2 * g + 1, prefetch=bufA)

    process(*bufA, NCH - 1, prefetch=None)

    plsc.subcore_barrier()

    def p2_chunk(out_ref, q):
      nb0 = n0 + q * PH
      pltpu.sync_copy(num_sp.at[pl.ds(nb0, PH), :], half_v)
      pltpu.sync_copy(den_sp.at[pl.ds(nb0, PH)], ea_v)

      @pl.loop(0, PH // 16)
      def _(g):
        sl = pl.ds(g * 16, 16)
        ea_v[sl] = 1.0 / (ea_v[sl] + EPS)

      @pl.loop(0, PH // 16)
      def _(g):
        ev = ea_v[pl.ds(g * 16, 16)]
        for j in range(16):
          i = g * 16 + j
          half_v[i, pl.ds(0, 16)] = half_v[i, pl.ds(0, 16)] * ev[j] + bH

      pltpu.sync_copy(half_v, out_ref.at[pl.ds(nb0, PH), :])

    for cc, ref in ((0, x3a_hbm), (1, x3b_hbm)):
      @pl.when(jnp.logical_and(c == cc, s < 15))
      def _(ref=ref):
        pl.loop(0, SLAB_A // PH)(functools.partial(p2_chunk, ref))

      @pl.when(jnp.logical_and(c == cc, s == 15))
      def _(ref=ref):
        pl.loop(0, SLAB_B // PH)(functools.partial(p2_chunk, ref))

  return k(src, dst, h2, atts2, attd2, b2f)


def _sc_gather_out(user_ids, item_ids, x3a, x3b):
  CH = 512

  mesh = plsc.VectorSubcoreMesh(**_MESH)

  @functools.partial(
      pl.kernel,
      out_type=tuple(
          jax.ShapeDtypeStruct((B, 16), jnp.float32) for _ in range(4)),
      mesh=mesh,
      compiler_params=_SC_PARAMS,
      scratch_types=(
          pltpu.VMEM((CH,), jnp.int32),
          pltpu.VMEM((CH, 16), jnp.float32),
          pltpu.SemaphoreType.DMA,
      ),
  )
  def k(uids_hbm, iids_hbm, x3a_hbm, x3b_hbm,
        ugl_hbm, ugr_hbm, igl_hbm, igr_hbm, idx_v, rows_v, sem):
    c = lax.axis_index("c")
    s = lax.axis_index("s")
    w = c * 16 + s
    b0 = w * CH

    pltpu.sync_copy(uids_hbm.at[pl.ds(b0, CH)], idx_v)
    pltpu.async_copy(x3a_hbm.at[idx_v], rows_v, sem).wait()
    pltpu.sync_copy(rows_v, ugl_hbm.at[pl.ds(b0, CH), :])
    pltpu.async_copy(x3b_hbm.at[idx_v], rows_v, sem).wait()
    pltpu.sync_copy(rows_v, ugr_hbm.at[pl.ds(b0, CH), :])

    pltpu.sync_copy(iids_hbm.at[pl.ds(b0, CH)], idx_v)

    @pl.loop(0, CH // 16)
    def _(g):
      sl = pl.ds(g * 16, 16)
      idx_v[sl] = idx_v[sl] + NU

    pltpu.async_copy(x3a_hbm.at[idx_v], rows_v, sem).wait()
    pltpu.sync_copy(rows_v, igl_hbm.at[pl.ds(b0, CH), :])
    pltpu.async_copy(x3b_hbm.at[idx_v], rows_v, sem).wait()
    pltpu.sync_copy(rows_v, igr_hbm.at[pl.ds(b0, CH), :])

  return k(user_ids, item_ids, x3a, x3b)


def kernel(user_ids, item_ids, content_features, edge_index, user_emb_table,
           item_emb_table, W_content, b_content, W1, att_src1, att_dst1, b1,
           W2, att_src2, att_dst2, b2, Wp1, bp1, Wp2, bp2):
  user_ids = user_ids.astype(jnp.int32)
  item_ids = item_ids.astype(jnp.int32)
  src = edge_index[0].astype(jnp.int32)
  dst = edge_index[1].astype(jnp.int32)

  c_emb = _tc_content(content_features, W_content.T,
                      b_content.reshape(1, 32))

  xL, xR, u_emb = _sc_build_x(user_ids, item_ids, c_emb,
                              user_emb_table, item_emb_table)

  W1st = jnp.stack([W1[:, :32], W1[:, 32:]])
  hh1p = _tc_hmatmul(xL.reshape(XP, 128), xR.reshape(XP, 128), W1st)
  hh1 = hh1p.reshape(2 * XROWS, 32)

  x2L, x2R, _, _ = _sc_edges1(src, dst, hh1, att_src1.reshape(64),
                              att_dst1.reshape(64), b1)

  h2p = _tc_hmatmul(x2L.reshape(XP, 128), x2R.reshape(XP, 128),
                    W2.reshape(1, 64, 32))
  h2 = h2p.reshape(XROWS, 32)

  x3a, x3b, _, _ = _sc_edges2(src, dst, h2, att_src2.reshape(32),
                              att_dst2.reshape(32), b2)

  ugL, ugR, igL, igR = _sc_gather_out(user_ids, item_ids, x3a, x3b)

  out = _tc_mlp(u_emb, igL, igR, ugL, ugR, Wp1.T, bp1.reshape(1, 32),
                Wp2.T, bp2.reshape(1, 1))
  return out.reshape(B)


# speedup vs baseline: 119.3965x; 1.1641x over previous
import functools

import jax
import jax.numpy as jnp
from jax import lax
from jax.experimental import pallas as pl
from jax.experimental.pallas import tpu as pltpu
from jax.experimental.pallas import tpu_sc as plsc

N = 50000
NU = 25000
B = 16384
E = 800000
XROWS = N + 16
XP = XROWS // 4
COFF = XROWS
EPS = 1e-16

_MESH = dict(core_axis_name="c", subcore_axis_name="s", num_cores=2,
             num_subcores=16)
_SC_PARAMS = pltpu.CompilerParams(needs_layout_passes=False,
                                  use_tc_tiling_on_sc=False)

SLAB_A = 3200
SLAB_B = 2000
PH = 400
EC = 400
LANE16 = None


def _tc_content(content, WcT, bc):
  r = 2048

  def body(c_ref, w_ref, b_ref, o_ref):
    o_ref[...] = (
        jnp.dot(c_ref[...], w_ref[...], preferred_element_type=jnp.float32)
        + b_ref[...])

  return pl.pallas_call(
      body,
      grid=(B // r,),
      in_specs=[
          pl.BlockSpec((r, 128), lambda i: (i, 0)),
          pl.BlockSpec((128, 32), lambda i: (0, 0)),
          pl.BlockSpec((1, 32), lambda i: (0, 0)),
      ],
      out_specs=pl.BlockSpec((r, 32), lambda i: (i, 0)),
      out_shape=jax.ShapeDtypeStruct((B, 32), jnp.float32),
  )(content, WcT, bc)


def _tc_hmatmul(xlp, xrp, Wst):
  R = xlp.shape[0]
  H = Wst.shape[0]
  nb = 3
  r = R // nb

  def body(xl_ref, xr_ref, w_ref, o_ref):
    parts = []
    for k in range(4):
      sl = slice(32 * k, 32 * (k + 1))
      hk = (jnp.dot(xl_ref[:, sl], w_ref[0, :32, :],
                    preferred_element_type=jnp.float32)
            + jnp.dot(xr_ref[:, sl], w_ref[0, 32:, :],
                      preferred_element_type=jnp.float32))
      parts.append(hk)
    o_ref[...] = jnp.concatenate(parts, axis=1)

  return pl.pallas_call(
      body,
      grid=(H, nb),
      in_specs=[
          pl.BlockSpec((r, 128), lambda c, i: (i, 0)),
          pl.BlockSpec((r, 128), lambda c, i: (i, 0)),
          pl.BlockSpec((1, 64, 32), lambda c, i: (c, 0, 0)),
      ],
      out_specs=pl.BlockSpec((r, 128), lambda c, i: (c * nb + i, 0)),
      out_shape=jax.ShapeDtypeStruct((H * R, 128), jnp.float32),
  )(xlp, xrp, Wst)


def _tc_mlp(u_emb, igL, igR, ugL, ugR, Wp1T, bp1v, Wp2T, bp2v):
  r = 2048

  def body(ue_ref, il_ref, ir_ref, ul_ref, ur_ref, w1_ref, b1_ref, w2_ref,
           b2_ref, o_ref):
    hdn = (jnp.dot(ue_ref[...], w1_ref[0:32, :],
                   preferred_element_type=jnp.float32)
           + jnp.dot(il_ref[...], w1_ref[32:48, :],
                     preferred_element_type=jnp.float32)
           + jnp.dot(ir_ref[...], w1_ref[48:64, :],
                     preferred_element_type=jnp.float32)
           + jnp.dot(ul_ref[...], w1_ref[64:80, :],
                     preferred_element_type=jnp.float32)
           + jnp.dot(ur_ref[...], w1_ref[80:96, :],
                     preferred_element_type=jnp.float32)
           + b1_ref[...])
    hdn = jnp.maximum(hdn, 0.0)
    o_ref[...] = (jnp.dot(hdn, w2_ref[...],
                          preferred_element_type=jnp.float32) + b2_ref[...])

  return pl.pallas_call(
      body,
      grid=(B // r,),
      in_specs=[
          pl.BlockSpec((r, 32), lambda i: (i, 0)),
          pl.BlockSpec((r, 16), lambda i: (i, 0)),
          pl.BlockSpec((r, 16), lambda i: (i, 0)),
          pl.BlockSpec((r, 16), lambda i: (i, 0)),
          pl.BlockSpec((r, 16), lambda i: (i, 0)),
          pl.BlockSpec((96, 32), lambda i: (0, 0)),
          pl.BlockSpec((1, 32), lambda i: (0, 0)),
          pl.BlockSpec((32, 1), lambda i: (0, 0)),
          pl.BlockSpec((1, 1), lambda i: (0, 0)),
      ],
      out_specs=pl.BlockSpec((r, 1), lambda i: (i, 0)),
      out_shape=jax.ShapeDtypeStruct((B, 1), jnp.float32),
  )(u_emb, igL, igR, ugL, ugR, Wp1T, bp1v, Wp2T, bp2v)


def _slabs(s):
  return s * SLAB_A, SLAB_A, SLAB_B


def _sc_build_x(user_ids, item_ids, c_emb, utab, itab):
  CH = 512
  ZR = 392

  mesh = plsc.VectorSubcoreMesh(**_MESH)

  @functools.partial(
      pl.kernel,
      out_type=(
          jax.ShapeDtypeStruct((XROWS, 32), jnp.float32),
          jax.ShapeDtypeStruct((XROWS, 32), jnp.float32),
          jax.ShapeDtypeStruct((B, 32), jnp.float32),
      ),
      mesh=mesh,
      compiler_params=_SC_PARAMS,
      scratch_types=(
          pltpu.VMEM((ZR, 32), jnp.float32),
          pltpu.VMEM((CH,), jnp.int32),
          pltpu.VMEM((CH,), jnp.int32),
          pltpu.VMEM((CH, 32), jnp.float32),
          pltpu.VMEM((CH, 32), jnp.float32),
          pltpu.VMEM((B,), jnp.int32),
          pltpu.VMEM((NU,), jnp.int32),
          pltpu.VMEM_SHARED((B,), jnp.int32),
          pltpu.SemaphoreType.DMA,
      ),
  )
  def k(uids_hbm, iids_hbm, cemb_hbm, utab_hbm, itab_hbm,
        xl_hbm, xr_hbm, uemb_hbm,
        zb, idx_v, tgt_v, rows_v, ce_v, iid_v, tab_v, tgt_sp, sem):
    c = lax.axis_index("c")
    s = lax.axis_index("s")

    zero16 = jnp.zeros((16,), jnp.float32)

    @pl.loop(0, ZR)
    def _(j):
      zb[j, pl.ds(0, 16)] = zero16
      zb[j, pl.ds(16, 16)] = zero16

    row0 = c * NU + s * 1568

    def zcopy(nr, koff):
      sl = pl.ds(row0 + koff, nr)
      pltpu.sync_copy(zb.at[pl.ds(0, nr), :], xl_hbm.at[sl, :])
      pltpu.sync_copy(zb.at[pl.ds(0, nr), :], xr_hbm.at[sl, :])

    for kk in range(3):
      zcopy(ZR, kk * ZR)

    @pl.when(s < 15)
    def _():
      zcopy(ZR, 3 * ZR)

    @pl.when(jnp.logical_and(s == 15, c == 0))
    def _():
      zcopy(304, 3 * ZR)

    @pl.when(jnp.logical_and(s == 15, c == 1))
    def _():
      zcopy(320, 3 * ZR)

    @pl.when(jnp.logical_and(c == 1, s == 0))
    def _():
      pltpu.sync_copy(iids_hbm, iid_v)

      lane = lax.iota(jnp.int32, 16)

      @pl.loop(0, B // 16)
      def _(g):
        ids = iid_v[pl.ds(g * 16, 16)]
        bv = lane + g * 16
        for j in range(16):
          plsc.store_scatter(tab_v, [ids], bv, mask=lane == j)

      @pl.loop(0, B // 16)
      def _(g):
        sl = pl.ds(g * 16, 16)
        bv = lane + g * 16
        ids = iid_v[sl]
        t = plsc.load_gather(tab_v, [ids])
        iid_v[sl] = jnp.where(t == bv, ids + NU, N + (bv & 15))

      pltpu.sync_copy(iid_v, tgt_sp)

    plsc.subcore_barrier()

    @pl.when(c == 0)
    def _():
      @pl.loop(0, 2)
      def _(kk):
        b0 = s * (2 * CH) + kk * CH
        pltpu.sync_copy(uids_hbm.at[pl.ds(b0, CH)], idx_v)
        pltpu.async_copy(utab_hbm.at[idx_v], rows_v, sem).wait()
        pltpu.sync_copy(rows_v, uemb_hbm.at[pl.ds(b0, CH), :])
        pltpu.sync_copy(rows_v, xl_hbm.at[idx_v])

    @pl.when(c == 1)
    def _():
      @pl.loop(0, 2)
      def _(kk):
        b0 = s * (2 * CH) + kk * CH
        pltpu.sync_copy(iids_hbm.at[pl.ds(b0, CH)], idx_v)
        pltpu.async_copy(itab_hbm.at[idx_v], rows_v, sem).wait()
        pltpu.sync_copy(tgt_sp.at[pl.ds(b0, CH)], tgt_v)
        pltpu.sync_copy(rows_v, xl_hbm.at[tgt_v])
        pltpu.sync_copy(cemb_hbm.at[pl.ds(b0, CH), :], ce_v)
        pltpu.sync_copy(ce_v, xr_hbm.at[tgt_v])

  return k(user_ids, item_ids, c_emb, utab, itab)


def _colgather_logits(rows_v, asA, asB, adA, adB, lane):
  accs = jnp.zeros((16,), jnp.float32)
  accd = jnp.zeros((16,), jnp.float32)
  for dcol in range(32):
    cvec = jnp.full((16,), dcol, jnp.int32)
    colv = plsc.load_gather(rows_v, [lane, cvec])
    sA = asA[dcol] if dcol < 16 else asB[dcol - 16]
    sD = adA[dcol] if dcol < 16 else adB[dcol - 16]
    accs = accs + colv * sA
    accd = accd + colv * sD
  return accs, accd


def _sc_edges1(src, dst, hh, attsf, attdf, b1f):
  EPT = E // 16
  NCH = EPT // EC

  mesh = plsc.VectorSubcoreMesh(**_MESH)

  @functools.partial(
      pl.kernel,
      out_type=(
          jax.ShapeDtypeStruct((XROWS, 32), jnp.float32),
          jax.ShapeDtypeStruct((XROWS, 32), jnp.float32),
          jax.ShapeDtypeStruct((2 * COFF,), jnp.float32),
          jax.ShapeDtypeStruct((2 * COFF,), jnp.float32),
      ),
      mesh=mesh,
      compiler_params=_SC_PARAMS,
      scratch_types=(
          pltpu.VMEM((EC,), jnp.int32),
          pltpu.VMEM((EC,), jnp.int32),
          pltpu.VMEM((EC,), jnp.int32),
          pltpu.VMEM((EC,), jnp.float32),
          pltpu.VMEM((EC,), jnp.float32),
          pltpu.VMEM((EC,), jnp.float32),
          pltpu.VMEM((EC,), jnp.int32),
          pltpu.VMEM((EC,), jnp.int32),
          pltpu.VMEM((EC,), jnp.float32),
          pltpu.VMEM((EC,), jnp.float32),
          pltpu.VMEM((EC, 32), jnp.float32),
          pltpu.VMEM((64,), jnp.float32),
          pltpu.VMEM((64,), jnp.float32),
          pltpu.VMEM((64,), jnp.float32),
          pltpu.VMEM_SHARED((N, 32), jnp.float32),
          pltpu.VMEM_SHARED((N,), jnp.float32),
          pltpu.SemaphoreType.DMA,
          pltpu.SemaphoreType.DMA,
          pltpu.SemaphoreType.DMA,
      ),
  )
  def k(src_hbm, dst_hbm, hh_hbm, atts_hbm, attd_hbm, b1_hbm,
        x2l_hbm, x2r_hbm, ahs_hbm, ahd_hbm,
        so_v, dst_v, do_v, ea_v, eb_v, ec_v, so2_v, do2_v, ea2_v, eb2_v,
        rows_v, atts_v, attd_v, b1_v,
        num_sp, den_sp, sem, sem2, sem3):
    c = lax.axis_index("c")
    s = lax.axis_index("s")
    coff = c * COFF
    lane = lax.iota(jnp.int32, 16)

    pltpu.sync_copy(atts_hbm, atts_v)
    pltpu.sync_copy(attd_hbm, attd_v)
    pltpu.sync_copy(b1_hbm, b1_v)
    is0 = c == 0
    asA = jnp.where(is0, atts_v[pl.ds(0, 16)], atts_v[pl.ds(32, 16)])
    asB = jnp.where(is0, atts_v[pl.ds(16, 16)], atts_v[pl.ds(48, 16)])
    adA = jnp.where(is0, attd_v[pl.ds(0, 16)], attd_v[pl.ds(32, 16)])
    adB = jnp.where(is0, attd_v[pl.ds(16, 16)], attd_v[pl.ds(48, 16)])
    bA = jnp.where(is0, b1_v[pl.ds(0, 16)], b1_v[pl.ds(32, 16)])
    bB = jnp.where(is0, b1_v[pl.ds(16, 16)], b1_v[pl.ds(48, 16)])

    n0 = s * SLAB_A

    def p0_chunk(q):
      nb0 = n0 + q * PH
      pltpu.sync_copy(hh_hbm.at[pl.ds(coff + nb0, PH), :], rows_v)

      @pl.loop(0, PH // 16)
      def _(g):
        lidx = lane + g * 16
        accs, accd = _colgather_logits(rows_v, asA, asB, adA, adB, lidx)
        al = accs + accd
        ex = jnp.exp(jnp.maximum(al, 0.2 * al))
        sl = pl.ds(g * 16, 16)
        ea_v[sl] = ex
        eb_v[sl] = accs
        ec_v[sl] = accd

      pltpu.sync_copy(eb_v, ahs_hbm.at[pl.ds(coff + nb0, PH)])
      pltpu.sync_copy(ec_v, ahd_hbm.at[pl.ds(coff + nb0, PH)])
      pltpu.sync_copy(ea_v, den_sp.at[pl.ds(nb0, PH)])

      @pl.loop(0, PH // 16)
      def _(g):
        ev = ea_v[pl.ds(g * 16, 16)]
        for j in range(16):
          i = g * 16 + j
          sc = ev[j]
          rows_v[i, pl.ds(0, 16)] = rows_v[i, pl.ds(0, 16)] * sc
          rows_v[i, pl.ds(16, 16)] = rows_v[i, pl.ds(16, 16)] * sc

      pltpu.sync_copy(rows_v, num_sp.at[pl.ds(nb0, PH), :])

    @pl.when(s < 15)
    def _():
      pl.loop(0, SLAB_A // PH)(p0_chunk)

    @pl.when(s == 15)
    def _():
      pl.loop(0, SLAB_B // PH)(p0_chunk)

    plsc.subcore_barrier()

    def issue(so, do, ea, eb, gsem, ch):
      e0 = s * EPT + ch * EC
      pltpu.sync_copy(src_hbm.at[pl.ds(e0, EC)], so)
      pltpu.sync_copy(dst_hbm.at[pl.ds(e0, EC)], do)

      @pl.loop(0, EC // 16)
      def _(g):
        sl = pl.ds(g * 16, 16)
        so[sl] = so[sl] + coff
        do[sl] = do[sl] + coff

      pltpu.async_copy(ahs_hbm.at[so], ea, gsem)
      pltpu.async_copy(ahd_hbm.at[do], eb, gsem)

    def process(so, do, ea, eb, gsem, ch, prefetch):
      pltpu.async_copy(hh_hbm.at[so], rows_v, sem2)
      if prefetch is not None:
        issue(*prefetch, ch + 1)
      pltpu.make_async_copy(ahs_hbm.at[so], ea, gsem).wait()
      pltpu.make_async_copy(ahd_hbm.at[do], eb, gsem).wait()

      @pl.loop(0, EC // 16)
      def _(g):
        sl = pl.ds(g * 16, 16)
        a = ea[sl] + eb[sl]
        ea[sl] = jnp.exp(jnp.maximum(a, 0.2 * a))
        dst_v[sl] = do[sl] - coff

      pltpu.sync_copy(ea, den_sp.at[dst_v], add=True)
      pltpu.make_async_copy(hh_hbm.at[so], rows_v, sem2).wait()

      @pl.loop(0, EC // 16)
      def _(g):
        ev = ea[pl.ds(g * 16, 16)]
        for j in range(16):
          i = g * 16 + j
          sc = ev[j]
          rows_v[i, pl.ds(0, 16)] = rows_v[i, pl.ds(0, 16)] * sc
          rows_v[i, pl.ds(16, 16)] = rows_v[i, pl.ds(16, 16)] * sc

      pltpu.sync_copy(rows_v, num_sp.at[dst_v], add=True)

    bufA = (so_v, do_v, ea_v, eb_v, sem)
    bufB = (so2_v, do2_v, ea2_v, eb2_v, sem3)
    issue(*bufA, 0)

    @pl.loop(0, (NCH - 1) // 2)
    def _(g):
      process(*bufA, 2 * g, prefetch=bufB)
      process(*bufB, 2 * g + 1, prefetch=bufA)

    process(*bufA, NCH - 1, prefetch=None)

    plsc.subcore_barrier()

    def p2_chunk(out_ref, q):
      nb0 = n0 + q * PH
      pltpu.sync_copy(num_sp.at[pl.ds(nb0, PH), :], rows_v)
      pltpu.sync_copy(den_sp.at[pl.ds(nb0, PH)], ea_v)

      @pl.loop(0, PH // 16)
      def _(g):
        sl = pl.ds(g * 16, 16)
        ea_v[sl] = 1.0 / (ea_v[sl] + EPS)

      @pl.loop(0, PH // 16)
      def _(g):
        ev = ea_v[pl.ds(g * 16, 16)]
        for j in range(16):
          i = g * 16 + j
          sc = ev[j]
          z0 = rows_v[i, pl.ds(0, 16)] * sc + bA
          z1 = rows_v[i, pl.ds(16, 16)] * sc + bB
          rows_v[i, pl.ds(0, 16)] = jnp.where(z0 > 0, z0, jnp.exp(z0) - 1.0)
          rows_v[i, pl.ds(16, 16)] = jnp.where(z1 > 0, z1, jnp.exp(z1) - 1.0)

      pltpu.sync_copy(rows_v, out_ref.at[pl.ds(nb0, PH), :])

    for cc, ref in ((0, x2l_hbm), (1, x2r_hbm)):
      @pl.when(jnp.logical_and(c == cc, s < 15))
      def _(ref=ref):
        pl.loop(0, SLAB_A // PH)(functools.partial(p2_chunk, ref))

      @pl.when(jnp.logical_and(c == cc, s == 15))
      def _(ref=ref):
        pl.loop(0, SLAB_B // PH)(functools.partial(p2_chunk, ref))

  return k(src, dst, hh, attsf, attdf, b1f)


def _sc_edges2(src, dst, h2, atts2, attd2, b2f):
  EPT = E // 16
  NCH = EPT // EC

  mesh = plsc.VectorSubcoreMesh(**_MESH)

  @functools.partial(
      pl.kernel,
      out_type=(
          jax.ShapeDtypeStruct((N, 16), jnp.float32),
          jax.ShapeDtypeStruct((N, 16), jnp.float32),
          jax.ShapeDtypeStruct((2 * N,), jnp.float32),
          jax.ShapeDtypeStruct((2 * N,), jnp.float32),
          jax.ShapeDtypeStruct((2 * N, 16), jnp.float32),
      ),
      mesh=mesh,
      compiler_params=_SC_PARAMS,
      scratch_types=(
          pltpu.VMEM((EC,), jnp.int32),
          pltpu.VMEM((EC,), jnp.int32),
          pltpu.VMEM((EC,), jnp.float32),
          pltpu.VMEM((EC,), jnp.float32),
          pltpu.VMEM((EC,), jnp.float32),
          pltpu.VMEM((EC,), jnp.int32),
          pltpu.VMEM((EC,), jnp.int32),
          pltpu.VMEM((EC,), jnp.float32),
          pltpu.VMEM((EC,), jnp.float32),
          pltpu.VMEM((EC, 32), jnp.float32),
          pltpu.VMEM((EC, 16), jnp.float32),
          pltpu.VMEM((32,), jnp.float32),
          pltpu.VMEM((32,), jnp.float32),
          pltpu.VMEM((32,), jnp.float32),
          pltpu.VMEM_SHARED((N, 16), jnp.float32),
          pltpu.VMEM_SHARED((N,), jnp.float32),
          pltpu.SemaphoreType.DMA,
          pltpu.SemaphoreType.DMA,
          pltpu.SemaphoreType.DMA,
      ),
  )
  def k(src_hbm, dst_hbm, h2_hbm, atts_hbm, attd_hbm, b2_hbm,
        x3a_hbm, x3b_hbm, ahs_hbm, ahd_hbm, h2cm_hbm,
        so_v, dst_v, ea_v, eb_v, ec_v, so2_v, dst2_v, ea2_v, eb2_v,
        rows_v, half_v, atts_v, attd_v, b2_v,
        num_sp, den_sp, sem, sem2, sem3):
    c = lax.axis_index("c")
    s = lax.axis_index("s")
    coff = c * N
    lane = lax.iota(jnp.int32, 16)

    pltpu.sync_copy(atts_hbm, atts_v)
    pltpu.sync_copy(attd_hbm, attd_v)
    pltpu.sync_copy(b2_hbm, b2_v)
    is0 = c == 0
    asA = atts_v[pl.ds(0, 16)]
    asB = atts_v[pl.ds(16, 16)]
    adA = attd_v[pl.ds(0, 16)]
    adB = attd_v[pl.ds(16, 16)]
    bH = jnp.where(is0, b2_v[pl.ds(0, 16)], b2_v[pl.ds(16, 16)])

    n0 = s * SLAB_A

    def half_row(i):
      r0 = rows_v[i, pl.ds(0, 16)]
      r1 = rows_v[i, pl.ds(16, 16)]
      return jnp.where(is0, r0, r1)

    def p0_chunk(q):
      nb0 = n0 + q * PH
      pltpu.sync_copy(h2_hbm.at[pl.ds(nb0, PH), :], rows_v)

      @pl.loop(0, PH // 16)
      def _(g):
        lidx = lane + g * 16
        accs, accd = _colgather_logits(rows_v, asA, asB, adA, adB, lidx)
        al = accs + accd
        ex = jnp.exp(jnp.maximum(al, 0.2 * al))
        sl = pl.ds(g * 16, 16)
        ea_v[sl] = ex
        eb_v[sl] = accs
        ec_v[sl] = accd

      pltpu.sync_copy(eb_v, ahs_hbm.at[pl.ds(coff + nb0, PH)])
      pltpu.sync_copy(ec_v, ahd_hbm.at[pl.ds(coff + nb0, PH)])
      pltpu.sync_copy(ea_v, den_sp.at[pl.ds(nb0, PH)])

      @pl.loop(0, PH // 16)
      def _(g):
        for j in range(16):
          i = g * 16 + j
          half_v[i, pl.ds(0, 16)] = half_row(i)

      pltpu.sync_copy(half_v, h2cm_hbm.at[pl.ds(coff + nb0, PH), :])

      @pl.loop(0, PH // 16)
      def _(g):
        ev = ea_v[pl.ds(g * 16, 16)]
        for j in range(16):
          i = g * 16 + j
          half_v[i, pl.ds(0, 16)] = half_v[i, pl.ds(0, 16)] * ev[j]

      pltpu.sync_copy(half_v, num_sp.at[pl.ds(nb0, PH), :])

    @pl.when(s < 15)
    def _():
      pl.loop(0, SLAB_A // PH)(p0_chunk)

    @pl.when(s == 15)
    def _():
      pl.loop(0, SLAB_B // PH)(p0_chunk)

    plsc.subcore_barrier()

    def issue(so, dv, ea, eb, gsem, ch):
      e0 = s * EPT + ch * EC
      pltpu.sync_copy(src_hbm.at[pl.ds(e0, EC)], so)
      pltpu.sync_copy(dst_hbm.at[pl.ds(e0, EC)], dv)

      @pl.loop(0, EC // 16)
      def _(g):
        sl = pl.ds(g * 16, 16)
        so[sl] = so[sl] + coff
        dv[sl] = dv[sl] + coff

      pltpu.async_copy(ahs_hbm.at[so], ea, gsem)
      pltpu.async_copy(ahd_hbm.at[dv], eb, gsem)

    def process(so, dv, ea, eb, gsem, ch, prefetch):
      pltpu.async_copy(h2cm_hbm.at[so], half_v, sem2)
      if prefetch is not None:
        issue(*prefetch, ch + 1)
      pltpu.make_async_copy(ahs_hbm.at[so], ea, gsem).wait()
      pltpu.make_async_copy(ahd_hbm.at[dv], eb, gsem).wait()

      @pl.loop(0, EC // 16)
      def _(g):
        sl = pl.ds(g * 16, 16)
        a = ea[sl] + eb[sl]
        ea[sl] = jnp.exp(jnp.maximum(a, 0.2 * a))
        dst_v2 = dv[sl] - coff
        dv[sl] = dst_v2

      pltpu.sync_copy(ea, den_sp.at[dv], add=True)
      pltpu.make_async_copy(h2cm_hbm.at[so], half_v, sem2).wait()

      @pl.loop(0, EC // 16)
      def _(g):
        ev = ea[pl.ds(g * 16, 16)]
        for j in range(16):
          i = g * 16 + j
          half_v[i, pl.ds(0, 16)] = half_v[i, pl.ds(0, 16)] * ev[j]

      pltpu.sync_copy(half_v, num_sp.at[dv], add=True)

    bufA = (so_v, dst_v, ea_v, eb_v, sem)
    bufB = (so2_v, dst2_v, ea2_v, eb2_v, sem3)
    issue(*bufA, 0)

    @pl.loop(0, (NCH - 1) // 2)
    def _(g):
      process(*bufA, 2 * g, prefetch=bufB)
      process(*bufB, 2 * g + 1, prefetch=bufA)

    process(*bufA, NCH - 1, prefetch=None)

    plsc.subcore_barrier()

    def p2_chunk(out_ref, q):
      nb0 = n0 + q * PH
      pltpu.sync_copy(num_sp.at[pl.ds(nb0, PH), :], half_v)
      pltpu.sync_copy(den_sp.at[pl.ds(nb0, PH)], ea_v)

      @pl.loop(0, PH // 16)
      def _(g):
        sl = pl.ds(g * 16, 16)
        ea_v[sl] = 1.0 / (ea_v[sl] + EPS)

      @pl.loop(0, PH // 16)
      def _(g):
        ev = ea_v[pl.ds(g * 16, 16)]
        for j in range(16):
          i = g * 16 + j
          half_v[i, pl.ds(0, 16)] = half_v[i, pl.ds(0, 16)] * ev[j] + bH

      pltpu.sync_copy(half_v, out_ref.at[pl.ds(nb0, PH), :])

    for cc, ref in ((0, x3a_hbm), (1, x3b_hbm)):
      @pl.when(jnp.logical_and(c == cc, s < 15))
      def _(ref=ref):
        pl.loop(0, SLAB_A // PH)(functools.partial(p2_chunk, ref))

      @pl.when(jnp.logical_and(c == cc, s == 15))
      def _(ref=ref):
        pl.loop(0, SLAB_B // PH)(functools.partial(p2_chunk, ref))

  return k(src, dst, h2, atts2, attd2, b2f)


def _sc_gather_out(user_ids, item_ids, x3a, x3b):
  CH = 512

  mesh = plsc.VectorSubcoreMesh(**_MESH)

  @functools.partial(
      pl.kernel,
      out_type=tuple(
          jax.ShapeDtypeStruct((B, 16), jnp.float32) for _ in range(4)),
      mesh=mesh,
      compiler_params=_SC_PARAMS,
      scratch_types=(
          pltpu.VMEM((CH,), jnp.int32),
          pltpu.VMEM((CH, 16), jnp.float32),
          pltpu.SemaphoreType.DMA,
      ),
  )
  def k(uids_hbm, iids_hbm, x3a_hbm, x3b_hbm,
        ugl_hbm, ugr_hbm, igl_hbm, igr_hbm, idx_v, rows_v, sem):
    c = lax.axis_index("c")
    s = lax.axis_index("s")
    w = c * 16 + s
    b0 = w * CH

    pltpu.sync_copy(uids_hbm.at[pl.ds(b0, CH)], idx_v)
    pltpu.async_copy(x3a_hbm.at[idx_v], rows_v, sem).wait()
    pltpu.sync_copy(rows_v, ugl_hbm.at[pl.ds(b0, CH), :])
    pltpu.async_copy(x3b_hbm.at[idx_v], rows_v, sem).wait()
    pltpu.sync_copy(rows_v, ugr_hbm.at[pl.ds(b0, CH), :])

    pltpu.sync_copy(iids_hbm.at[pl.ds(b0, CH)], idx_v)

    @pl.loop(0, CH // 16)
    def _(g):
      sl = pl.ds(g * 16, 16)
      idx_v[sl] = idx_v[sl] + NU

    pltpu.async_copy(x3a_hbm.at[idx_v], rows_v, sem).wait()
    pltpu.sync_copy(rows_v, igl_hbm.at[pl.ds(b0, CH), :])
    pltpu.async_copy(x3b_hbm.at[idx_v], rows_v, sem).wait()
    pltpu.sync_copy(rows_v, igr_hbm.at[pl.ds(b0, CH), :])

  return k(user_ids, item_ids, x3a, x3b)


def kernel(user_ids, item_ids, content_features, edge_index, user_emb_table,
           item_emb_table, W_content, b_content, W1, att_src1, att_dst1, b1,
           W2, att_src2, att_dst2, b2, Wp1, bp1, Wp2, bp2):
  user_ids = user_ids.astype(jnp.int32)
  item_ids = item_ids.astype(jnp.int32)
  src = edge_index[0].astype(jnp.int32)
  dst = edge_index[1].astype(jnp.int32)

  c_emb = _tc_content(content_features, W_content.T,
                      b_content.reshape(1, 32))

  xL, xR, u_emb = _sc_build_x(user_ids, item_ids, c_emb,
                              user_emb_table, item_emb_table)

  W1st = jnp.stack([W1[:, :32], W1[:, 32:]])
  hh1p = _tc_hmatmul(xL.reshape(XP, 128), xR.reshape(XP, 128), W1st)
  hh1 = hh1p.reshape(2 * XROWS, 32)

  x2L, x2R, _, _ = _sc_edges1(src, dst, hh1, att_src1.reshape(64),
                              att_dst1.reshape(64), b1)

  h2p = _tc_hmatmul(x2L.reshape(XP, 128), x2R.reshape(XP, 128),
                    W2.reshape(1, 64, 32))
  h2 = h2p.reshape(XROWS, 32)

  x3a, x3b, _, _, _ = _sc_edges2(src, dst, h2, att_src2.reshape(32),
                                 att_dst2.reshape(32), b2)

  ugL, ugR, igL, igR = _sc_gather_out(user_ids, item_ids, x3a, x3b)

  out = _tc_mlp(u_emb, igL, igR, ugL, ugR, Wp1.T, bp1.reshape(1, 32),
                Wp2.T, bp2.reshape(1, 1))
  return out.reshape(B)


# speedup vs baseline: 131.1978x; 1.0988x over previous
import functools

import jax
import jax.numpy as jnp
from jax import lax
from jax.experimental import pallas as pl
from jax.experimental.pallas import tpu as pltpu
from jax.experimental.pallas import tpu_sc as plsc

N = 50000
NU = 25000
B = 16384
E = 800000
XROWS = N + 16
XP = XROWS // 4
COFF = XROWS
EPS = 1e-16

_MESH = dict(core_axis_name="c", subcore_axis_name="s", num_cores=2,
             num_subcores=16)
_SC_PARAMS = pltpu.CompilerParams(needs_layout_passes=False,
                                  use_tc_tiling_on_sc=False)

SLAB_A = 3200
SLAB_B = 2000
PH = 400
EC = 400
LANE16 = None


def _tc_content(content, WcT, bc):
  r = 2048

  def body(c_ref, w_ref, b_ref, o_ref):
    o_ref[...] = (
        jnp.dot(c_ref[...], w_ref[...], preferred_element_type=jnp.float32)
        + b_ref[...])

  return pl.pallas_call(
      body,
      grid=(B // r,),
      in_specs=[
          pl.BlockSpec((r, 128), lambda i: (i, 0)),
          pl.BlockSpec((128, 32), lambda i: (0, 0)),
          pl.BlockSpec((1, 32), lambda i: (0, 0)),
      ],
      out_specs=pl.BlockSpec((r, 32), lambda i: (i, 0)),
      out_shape=jax.ShapeDtypeStruct((B, 32), jnp.float32),
  )(content, WcT, bc)


def _tc_hmatmul(xlp, xrp, Wst):
  R = xlp.shape[0]
  H = Wst.shape[0]
  nb = 3
  r = R // nb

  def body(xl_ref, xr_ref, w_ref, o_ref):
    parts = []
    for k in range(4):
      sl = slice(32 * k, 32 * (k + 1))
      hk = (jnp.dot(xl_ref[:, sl], w_ref[0, :32, :],
                    preferred_element_type=jnp.float32)
            + jnp.dot(xr_ref[:, sl], w_ref[0, 32:, :],
                      preferred_element_type=jnp.float32))
      parts.append(hk)
    o_ref[...] = jnp.concatenate(parts, axis=1)

  return pl.pallas_call(
      body,
      grid=(H, nb),
      in_specs=[
          pl.BlockSpec((r, 128), lambda c, i: (i, 0)),
          pl.BlockSpec((r, 128), lambda c, i: (i, 0)),
          pl.BlockSpec((1, 64, 32), lambda c, i: (c, 0, 0)),
      ],
      out_specs=pl.BlockSpec((r, 128), lambda c, i: (c * nb + i, 0)),
      out_shape=jax.ShapeDtypeStruct((H * R, 128), jnp.float32),
  )(xlp, xrp, Wst)


def _tc_mlp(u_emb, igL, igR, ugL, ugR, Wp1T, bp1v, Wp2T, bp2v):
  r = 2048

  def body(ue_ref, il_ref, ir_ref, ul_ref, ur_ref, w1_ref, b1_ref, w2_ref,
           b2_ref, o_ref):
    hdn = (jnp.dot(ue_ref[...], w1_ref[0:32, :],
                   preferred_element_type=jnp.float32)
           + jnp.dot(il_ref[...], w1_ref[32:48, :],
                     preferred_element_type=jnp.float32)
           + jnp.dot(ir_ref[...], w1_ref[48:64, :],
                     preferred_element_type=jnp.float32)
           + jnp.dot(ul_ref[...], w1_ref[64:80, :],
                     preferred_element_type=jnp.float32)
           + jnp.dot(ur_ref[...], w1_ref[80:96, :],
                     preferred_element_type=jnp.float32)
           + b1_ref[...])
    hdn = jnp.maximum(hdn, 0.0)
    o_ref[...] = (jnp.dot(hdn, w2_ref[...],
                          preferred_element_type=jnp.float32) + b2_ref[...])

  return pl.pallas_call(
      body,
      grid=(B // r,),
      in_specs=[
          pl.BlockSpec((r, 32), lambda i: (i, 0)),
          pl.BlockSpec((r, 16), lambda i: (i, 0)),
          pl.BlockSpec((r, 16), lambda i: (i, 0)),
          pl.BlockSpec((r, 16), lambda i: (i, 0)),
          pl.BlockSpec((r, 16), lambda i: (i, 0)),
          pl.BlockSpec((96, 32), lambda i: (0, 0)),
          pl.BlockSpec((1, 32), lambda i: (0, 0)),
          pl.BlockSpec((32, 1), lambda i: (0, 0)),
          pl.BlockSpec((1, 1), lambda i: (0, 0)),
      ],
      out_specs=pl.BlockSpec((r, 1), lambda i: (i, 0)),
      out_shape=jax.ShapeDtypeStruct((B, 1), jnp.float32),
  )(u_emb, igL, igR, ugL, ugR, Wp1T, bp1v, Wp2T, bp2v)


def _slabs(s):
  return s * SLAB_A, SLAB_A, SLAB_B


def _sc_build_x(user_ids, item_ids, c_emb, utab, itab):
  CH = 512
  ZR = 392

  mesh = plsc.VectorSubcoreMesh(**_MESH)

  @functools.partial(
      pl.kernel,
      out_type=(
          jax.ShapeDtypeStruct((XROWS, 32), jnp.float32),
          jax.ShapeDtypeStruct((XROWS, 32), jnp.float32),
          jax.ShapeDtypeStruct((B, 32), jnp.float32),
      ),
      mesh=mesh,
      compiler_params=_SC_PARAMS,
      scratch_types=(
          pltpu.VMEM((ZR, 32), jnp.float32),
          pltpu.VMEM((CH,), jnp.int32),
          pltpu.VMEM((CH,), jnp.int32),
          pltpu.VMEM((CH, 32), jnp.float32),
          pltpu.VMEM((CH, 32), jnp.float32),
          pltpu.VMEM((B,), jnp.int32),
          pltpu.VMEM((NU,), jnp.int32),
          pltpu.VMEM_SHARED((B,), jnp.int32),
          pltpu.SemaphoreType.DMA,
      ),
  )
  def k(uids_hbm, iids_hbm, cemb_hbm, utab_hbm, itab_hbm,
        xl_hbm, xr_hbm, uemb_hbm,
        zb, idx_v, tgt_v, rows_v, ce_v, iid_v, tab_v, tgt_sp, sem):
    c = lax.axis_index("c")
    s = lax.axis_index("s")

    zero16 = jnp.zeros((16,), jnp.float32)

    @pl.loop(0, ZR)
    def _(j):
      zb[j, pl.ds(0, 16)] = zero16
      zb[j, pl.ds(16, 16)] = zero16

    row0 = c * NU + s * 1568

    def zcopy(nr, koff):
      sl = pl.ds(row0 + koff, nr)
      pltpu.sync_copy(zb.at[pl.ds(0, nr), :], xl_hbm.at[sl, :])
      pltpu.sync_copy(zb.at[pl.ds(0, nr), :], xr_hbm.at[sl, :])

    for kk in range(3):
      zcopy(ZR, kk * ZR)

    @pl.when(s < 15)
    def _():
      zcopy(ZR, 3 * ZR)

    @pl.when(jnp.logical_and(s == 15, c == 0))
    def _():
      zcopy(304, 3 * ZR)

    @pl.when(jnp.logical_and(s == 15, c == 1))
    def _():
      zcopy(320, 3 * ZR)

    @pl.when(jnp.logical_and(c == 1, s == 0))
    def _():
      pltpu.sync_copy(iids_hbm, iid_v)

      lane = lax.iota(jnp.int32, 16)

      @pl.loop(0, B // 16)
      def _(g):
        ids = iid_v[pl.ds(g * 16, 16)]
        bv = lane + g * 16
        for j in range(16):
          plsc.store_scatter(tab_v, [ids], bv, mask=lane == j)

      @pl.loop(0, B // 16)
      def _(g):
        sl = pl.ds(g * 16, 16)
        bv = lane + g * 16
        ids = iid_v[sl]
        t = plsc.load_gather(tab_v, [ids])
        iid_v[sl] = jnp.where(t == bv, ids + NU, N + (bv & 15))

      pltpu.sync_copy(iid_v, tgt_sp)

    plsc.subcore_barrier()

    @pl.when(c == 0)
    def _():
      @pl.loop(0, 2)
      def _(kk):
        b0 = s * (2 * CH) + kk * CH
        pltpu.sync_copy(uids_hbm.at[pl.ds(b0, CH)], idx_v)
        pltpu.async_copy(utab_hbm.at[idx_v], rows_v, sem).wait()
        pltpu.sync_copy(rows_v, uemb_hbm.at[pl.ds(b0, CH), :])
        pltpu.sync_copy(rows_v, xl_hbm.at[idx_v])

    @pl.when(c == 1)
    def _():
      @pl.loop(0, 2)
      def _(kk):
        b0 = s * (2 * CH) + kk * CH
        pltpu.sync_copy(iids_hbm.at[pl.ds(b0, CH)], idx_v)
        pltpu.async_copy(itab_hbm.at[idx_v], rows_v, sem).wait()
        pltpu.sync_copy(tgt_sp.at[pl.ds(b0, CH)], tgt_v)
        pltpu.sync_copy(rows_v, xl_hbm.at[tgt_v])
        pltpu.sync_copy(cemb_hbm.at[pl.ds(b0, CH), :], ce_v)
        pltpu.sync_copy(ce_v, xr_hbm.at[tgt_v])

  return k(user_ids, item_ids, c_emb, utab, itab)


def _colgather_logits(rows_v, asA, asB, adA, adB, lane):
  accs = jnp.zeros((16,), jnp.float32)
  accd = jnp.zeros((16,), jnp.float32)
  for dcol in range(32):
    cvec = jnp.full((16,), dcol, jnp.int32)
    colv = plsc.load_gather(rows_v, [lane, cvec])
    sA = asA[dcol] if dcol < 16 else asB[dcol - 16]
    sD = adA[dcol] if dcol < 16 else adB[dcol - 16]
    accs = accs + colv * sA
    accd = accd + colv * sD
  return accs, accd


def _sc_edges1(src, dst, hh, attsf, attdf, b1f):
  EPT = E // 16
  NCH = EPT // EC

  mesh = plsc.VectorSubcoreMesh(**_MESH)

  @functools.partial(
      pl.kernel,
      out_type=(
          jax.ShapeDtypeStruct((XROWS, 32), jnp.float32),
          jax.ShapeDtypeStruct((XROWS, 32), jnp.float32),
          jax.ShapeDtypeStruct((2 * COFF,), jnp.float32),
          jax.ShapeDtypeStruct((2 * COFF,), jnp.float32),
      ),
      mesh=mesh,
      compiler_params=_SC_PARAMS,
      scratch_types=(
          pltpu.VMEM((EC,), jnp.int32),
          pltpu.VMEM((EC,), jnp.int32),
          pltpu.VMEM((EC,), jnp.int32),
          pltpu.VMEM((EC,), jnp.float32),
          pltpu.VMEM((EC,), jnp.float32),
          pltpu.VMEM((EC,), jnp.float32),
          pltpu.VMEM((EC,), jnp.int32),
          pltpu.VMEM((EC,), jnp.int32),
          pltpu.VMEM((EC,), jnp.float32),
          pltpu.VMEM((EC,), jnp.float32),
          pltpu.VMEM((EC, 32), jnp.float32),
          pltpu.VMEM((64,), jnp.float32),
          pltpu.VMEM((64,), jnp.float32),
          pltpu.VMEM((64,), jnp.float32),
          pltpu.VMEM_SHARED((N, 32), jnp.float32),
          pltpu.VMEM_SHARED((N,), jnp.float32),
          pltpu.SemaphoreType.DMA,
          pltpu.SemaphoreType.DMA,
          pltpu.SemaphoreType.DMA,
      ),
  )
  def k(src_hbm, dst_hbm, hh_hbm, atts_hbm, attd_hbm, b1_hbm,
        x2l_hbm, x2r_hbm, ahs_hbm, ahd_hbm,
        so_v, dst_v, do_v, ea_v, eb_v, ec_v, so2_v, do2_v, ea2_v, eb2_v,
        rows_v, atts_v, attd_v, b1_v,
        num_sp, den_sp, sem, sem2, sem3):
    c = lax.axis_index("c")
    s = lax.axis_index("s")
    coff = c * COFF
    lane = lax.iota(jnp.int32, 16)

    pltpu.sync_copy(atts_hbm, atts_v)
    pltpu.sync_copy(attd_hbm, attd_v)
    pltpu.sync_copy(b1_hbm, b1_v)
    is0 = c == 0
    asA = jnp.where(is0, atts_v[pl.ds(0, 16)], atts_v[pl.ds(32, 16)])
    asB = jnp.where(is0, atts_v[pl.ds(16, 16)], atts_v[pl.ds(48, 16)])
    adA = jnp.where(is0, attd_v[pl.ds(0, 16)], attd_v[pl.ds(32, 16)])
    adB = jnp.where(is0, attd_v[pl.ds(16, 16)], attd_v[pl.ds(48, 16)])
    bA = jnp.where(is0, b1_v[pl.ds(0, 16)], b1_v[pl.ds(32, 16)])
    bB = jnp.where(is0, b1_v[pl.ds(16, 16)], b1_v[pl.ds(48, 16)])

    n0 = s * SLAB_A

    def p0_chunk(q):
      nb0 = n0 + q * PH
      pltpu.sync_copy(hh_hbm.at[pl.ds(coff + nb0, PH), :], rows_v)

      @pl.loop(0, PH // 16)
      def _(g):
        lidx = lane + g * 16
        accs, accd = _colgather_logits(rows_v, asA, asB, adA, adB, lidx)
        al = accs + accd
        ex = jnp.exp(jnp.maximum(al, 0.2 * al))
        sl = pl.ds(g * 16, 16)
        ea_v[sl] = ex
        eb_v[sl] = accs
        ec_v[sl] = accd

      pltpu.sync_copy(eb_v, ahs_hbm.at[pl.ds(coff + nb0, PH)])
      pltpu.sync_copy(ec_v, ahd_hbm.at[pl.ds(coff + nb0, PH)])
      pltpu.sync_copy(ea_v, den_sp.at[pl.ds(nb0, PH)])

      @pl.loop(0, PH // 16)
      def _(g):
        ev = ea_v[pl.ds(g * 16, 16)]
        for j in range(16):
          i = g * 16 + j
          sc = ev[j]
          rows_v[i, pl.ds(0, 16)] = rows_v[i, pl.ds(0, 16)] * sc
          rows_v[i, pl.ds(16, 16)] = rows_v[i, pl.ds(16, 16)] * sc

      pltpu.sync_copy(rows_v, num_sp.at[pl.ds(nb0, PH), :])

    @pl.when(s < 15)
    def _():
      pl.loop(0, SLAB_A // PH)(p0_chunk)

    @pl.when(s == 15)
    def _():
      pl.loop(0, SLAB_B // PH)(p0_chunk)

    plsc.subcore_barrier()

    def issue(so, do, ea, eb, gsem, ch):
      e0 = s * EPT + ch * EC
      pltpu.sync_copy(src_hbm.at[pl.ds(e0, EC)], so)
      pltpu.sync_copy(dst_hbm.at[pl.ds(e0, EC)], do)

      @pl.loop(0, EC // 16)
      def _(g):
        sl = pl.ds(g * 16, 16)
        so[sl] = so[sl] + coff
        do[sl] = do[sl] + coff

      pltpu.async_copy(ahs_hbm.at[so], ea, gsem)
      pltpu.async_copy(ahd_hbm.at[do], eb, gsem)

    def process(so, do, ea, eb, gsem, ch, prefetch):
      pltpu.async_copy(hh_hbm.at[so], rows_v, sem2)
      if prefetch is not None:
        issue(*prefetch, ch + 1)
      pltpu.make_async_copy(ahs_hbm.at[so], ea, gsem).wait()
      pltpu.make_async_copy(ahd_hbm.at[do], eb, gsem).wait()

      @pl.loop(0, EC // 16)
      def _(g):
        sl = pl.ds(g * 16, 16)
        a = ea[sl] + eb[sl]
        ea[sl] = jnp.exp(jnp.maximum(a, 0.2 * a))
        dst_v[sl] = do[sl] - coff

      pltpu.sync_copy(ea, den_sp.at[dst_v], add=True)
      pltpu.make_async_copy(hh_hbm.at[so], rows_v, sem2).wait()

      @pl.loop(0, EC // 16)
      def _(g):
        ev = ea[pl.ds(g * 16, 16)]
        for j in range(16):
          i = g * 16 + j
          sc = ev[j]
          rows_v[i, pl.ds(0, 16)] = rows_v[i, pl.ds(0, 16)] * sc
          rows_v[i, pl.ds(16, 16)] = rows_v[i, pl.ds(16, 16)] * sc

      pltpu.sync_copy(rows_v, num_sp.at[dst_v], add=True)

    bufA = (so_v, do_v, ea_v, eb_v, sem)
    bufB = (so2_v, do2_v, ea2_v, eb2_v, sem3)
    issue(*bufA, 0)

    @pl.loop(0, (NCH - 1) // 2)
    def _(g):
      process(*bufA, 2 * g, prefetch=bufB)
      process(*bufB, 2 * g + 1, prefetch=bufA)

    process(*bufA, NCH - 1, prefetch=None)

    plsc.subcore_barrier()

    def p2_chunk(out_ref, q):
      nb0 = n0 + q * PH
      pltpu.sync_copy(num_sp.at[pl.ds(nb0, PH), :], rows_v)
      pltpu.sync_copy(den_sp.at[pl.ds(nb0, PH)], ea_v)

      @pl.loop(0, PH // 16)
      def _(g):
        sl = pl.ds(g * 16, 16)
        ea_v[sl] = 1.0 / (ea_v[sl] + EPS)

      @pl.loop(0, PH // 16)
      def _(g):
        ev = ea_v[pl.ds(g * 16, 16)]
        for j in range(16):
          i = g * 16 + j
          sc = ev[j]
          z0 = rows_v[i, pl.ds(0, 16)] * sc + bA
          z1 = rows_v[i, pl.ds(16, 16)] * sc + bB
          rows_v[i, pl.ds(0, 16)] = jnp.where(z0 > 0, z0, jnp.exp(z0) - 1.0)
          rows_v[i, pl.ds(16, 16)] = jnp.where(z1 > 0, z1, jnp.exp(z1) - 1.0)

      pltpu.sync_copy(rows_v, out_ref.at[pl.ds(nb0, PH), :])

    for cc, ref in ((0, x2l_hbm), (1, x2r_hbm)):
      @pl.when(jnp.logical_and(c == cc, s < 15))
      def _(ref=ref):
        pl.loop(0, SLAB_A // PH)(functools.partial(p2_chunk, ref))

      @pl.when(jnp.logical_and(c == cc, s == 15))
      def _(ref=ref):
        pl.loop(0, SLAB_B // PH)(functools.partial(p2_chunk, ref))

  return k(src, dst, hh, attsf, attdf, b1f)


def _sc_edges2(src, dst, h2, atts2, attd2, b2f):
  EC2 = 2000
  EPT = E // 16
  NCH = EPT // EC2

  mesh = plsc.VectorSubcoreMesh(**_MESH)

  @functools.partial(
      pl.kernel,
      out_type=(
          jax.ShapeDtypeStruct((N, 16), jnp.float32),
          jax.ShapeDtypeStruct((N, 16), jnp.float32),
          jax.ShapeDtypeStruct((2 * N,), jnp.float32),
          jax.ShapeDtypeStruct((2 * N,), jnp.float32),
          jax.ShapeDtypeStruct((2 * N, 16), jnp.float32),
      ),
      mesh=mesh,
      compiler_params=_SC_PARAMS,
      scratch_types=(
          pltpu.VMEM((EC2,), jnp.int32),
          pltpu.VMEM((EC2,), jnp.int32),
          pltpu.VMEM((EC2,), jnp.float32),
          pltpu.VMEM((EC2,), jnp.float32),
          pltpu.VMEM((PH,), jnp.float32),
          pltpu.VMEM((EC2,), jnp.int32),
          pltpu.VMEM((EC2,), jnp.int32),
          pltpu.VMEM((EC2,), jnp.float32),
          pltpu.VMEM((EC2,), jnp.float32),
          pltpu.VMEM((PH, 32), jnp.float32),
          pltpu.VMEM((EC2, 16), jnp.float32),
          pltpu.VMEM((32,), jnp.float32),
          pltpu.VMEM((32,), jnp.float32),
          pltpu.VMEM((32,), jnp.float32),
          pltpu.VMEM_SHARED((N, 16), jnp.float32),
          pltpu.VMEM_SHARED((N,), jnp.float32),
          pltpu.SemaphoreType.DMA,
          pltpu.SemaphoreType.DMA,
          pltpu.SemaphoreType.DMA,
      ),
  )
  def k(src_hbm, dst_hbm, h2_hbm, atts_hbm, attd_hbm, b2_hbm,
        x3a_hbm, x3b_hbm, ahs_hbm, ahd_hbm, h2cm_hbm,
        so_v, dst_v, ea_v, eb_v, ec_v, so2_v, dst2_v, ea2_v, eb2_v,
        rows_v, half_v, atts_v, attd_v, b2_v,
        num_sp, den_sp, sem, sem2, sem3):
    c = lax.axis_index("c")
    s = lax.axis_index("s")
    coff = c * N
    lane = lax.iota(jnp.int32, 16)

    pltpu.sync_copy(atts_hbm, atts_v)
    pltpu.sync_copy(attd_hbm, attd_v)
    pltpu.sync_copy(b2_hbm, b2_v)
    is0 = c == 0
    asA = atts_v[pl.ds(0, 16)]
    asB = atts_v[pl.ds(16, 16)]
    adA = attd_v[pl.ds(0, 16)]
    adB = attd_v[pl.ds(16, 16)]
    bH = jnp.where(is0, b2_v[pl.ds(0, 16)], b2_v[pl.ds(16, 16)])

    n0 = s * SLAB_A

    def half_row(i):
      r0 = rows_v[i, pl.ds(0, 16)]
      r1 = rows_v[i, pl.ds(16, 16)]
      return jnp.where(is0, r0, r1)

    def p0_chunk(q):
      nb0 = n0 + q * PH
      pltpu.sync_copy(h2_hbm.at[pl.ds(nb0, PH), :], rows_v)

      @pl.loop(0, PH // 16)
      def _(g):
        lidx = lane + g * 16
        accs, accd = _colgather_logits(rows_v, asA, asB, adA, adB, lidx)
        al = accs + accd
        ex = jnp.exp(jnp.maximum(al, 0.2 * al))
        sl = pl.ds(g * 16, 16)
        ea_v[sl] = ex
        eb_v[sl] = accs
        ec_v[sl] = accd

      pltpu.sync_copy(eb_v.at[pl.ds(0, PH)], ahs_hbm.at[pl.ds(coff + nb0, PH)])
      pltpu.sync_copy(ec_v, ahd_hbm.at[pl.ds(coff + nb0, PH)])
      pltpu.sync_copy(ea_v.at[pl.ds(0, PH)], den_sp.at[pl.ds(nb0, PH)])

      @pl.loop(0, PH // 16)
      def _(g):
        for j in range(16):
          i = g * 16 + j
          half_v[i, pl.ds(0, 16)] = half_row(i)

      pltpu.sync_copy(half_v.at[pl.ds(0, PH), :],
                      h2cm_hbm.at[pl.ds(coff + nb0, PH), :])

      @pl.loop(0, PH // 16)
      def _(g):
        ev = ea_v[pl.ds(g * 16, 16)]
        for j in range(16):
          i = g * 16 + j
          half_v[i, pl.ds(0, 16)] = half_v[i, pl.ds(0, 16)] * ev[j]

      pltpu.sync_copy(half_v.at[pl.ds(0, PH), :],
                      num_sp.at[pl.ds(nb0, PH), :])

    @pl.when(s < 15)
    def _():
      pl.loop(0, SLAB_A // PH)(p0_chunk)

    @pl.when(s == 15)
    def _():
      pl.loop(0, SLAB_B // PH)(p0_chunk)

    plsc.subcore_barrier()

    def issue(so, dv, ea, eb, gsem, ch):
      e0 = s * EPT + ch * EC2
      pltpu.sync_copy(src_hbm.at[pl.ds(e0, EC2)], so)
      pltpu.sync_copy(dst_hbm.at[pl.ds(e0, EC2)], dv)

      @pl.loop(0, EC2 // 16)
      def _(g):
        sl = pl.ds(g * 16, 16)
        so[sl] = so[sl] + coff
        dv[sl] = dv[sl] + coff

      pltpu.async_copy(ahs_hbm.at[so], ea, gsem)
      pltpu.async_copy(ahd_hbm.at[dv], eb, gsem)

    def process(so, dv, ea, eb, gsem, ch, prefetch):
      pltpu.async_copy(h2cm_hbm.at[so], half_v, sem2)
      if prefetch is not None:
        issue(*prefetch, ch + 1)
      pltpu.make_async_copy(ahs_hbm.at[so], ea, gsem).wait()
      pltpu.make_async_copy(ahd_hbm.at[dv], eb, gsem).wait()

      @pl.loop(0, EC2 // 16)
      def _(g):
        sl = pl.ds(g * 16, 16)
        a = ea[sl] + eb[sl]
        ea[sl] = jnp.exp(jnp.maximum(a, 0.2 * a))
        dst_v2 = dv[sl] - coff
        dv[sl] = dst_v2

      pltpu.sync_copy(ea, den_sp.at[dv], add=True)
      pltpu.make_async_copy(h2cm_hbm.at[so], half_v, sem2).wait()

      @pl.loop(0, EC2 // 16)
      def _(g):
        ev = ea[pl.ds(g * 16, 16)]
        for j in range(16):
          i = g * 16 + j
          half_v[i, pl.ds(0, 16)] = half_v[i, pl.ds(0, 16)] * ev[j]

      pltpu.sync_copy(half_v, num_sp.at[dv], add=True)

    bufA = (so_v, dst_v, ea_v, eb_v, sem)
    bufB = (so2_v, dst2_v, ea2_v, eb2_v, sem3)
    issue(*bufA, 0)

    @pl.loop(0, (NCH - 1) // 2)
    def _(g):
      process(*bufA, 2 * g, prefetch=bufB)
      process(*bufB, 2 * g + 1, prefetch=bufA)

    process(*bufA, NCH - 1, prefetch=None)

    plsc.subcore_barrier()

    def p2_chunk(out_ref, q):
      nb0 = n0 + q * PH
      pltpu.sync_copy(num_sp.at[pl.ds(nb0, PH), :], half_v.at[pl.ds(0, PH), :])
      pltpu.sync_copy(den_sp.at[pl.ds(nb0, PH)], ea_v.at[pl.ds(0, PH)])

      @pl.loop(0, PH // 16)
      def _(g):
        sl = pl.ds(g * 16, 16)
        ea_v[sl] = 1.0 / (ea_v[sl] + EPS)

      @pl.loop(0, PH // 16)
      def _(g):
        ev = ea_v[pl.ds(g * 16, 16)]
        for j in range(16):
          i = g * 16 + j
          half_v[i, pl.ds(0, 16)] = half_v[i, pl.ds(0, 16)] * ev[j] + bH

      pltpu.sync_copy(half_v.at[pl.ds(0, PH), :], out_ref.at[pl.ds(nb0, PH), :])

    for cc, ref in ((0, x3a_hbm), (1, x3b_hbm)):
      @pl.when(jnp.logical_and(c == cc, s < 15))
      def _(ref=ref):
        pl.loop(0, SLAB_A // PH)(functools.partial(p2_chunk, ref))

      @pl.when(jnp.logical_and(c == cc, s == 15))
      def _(ref=ref):
        pl.loop(0, SLAB_B // PH)(functools.partial(p2_chunk, ref))

  return k(src, dst, h2, atts2, attd2, b2f)


def _sc_gather_out(user_ids, item_ids, x3a, x3b):
  CH = 512

  mesh = plsc.VectorSubcoreMesh(**_MESH)

  @functools.partial(
      pl.kernel,
      out_type=tuple(
          jax.ShapeDtypeStruct((B, 16), jnp.float32) for _ in range(4)),
      mesh=mesh,
      compiler_params=_SC_PARAMS,
      scratch_types=(
          pltpu.VMEM((CH,), jnp.int32),
          pltpu.VMEM((CH, 16), jnp.float32),
          pltpu.SemaphoreType.DMA,
      ),
  )
  def k(uids_hbm, iids_hbm, x3a_hbm, x3b_hbm,
        ugl_hbm, ugr_hbm, igl_hbm, igr_hbm, idx_v, rows_v, sem):
    c = lax.axis_index("c")
    s = lax.axis_index("s")
    w = c * 16 + s
    b0 = w * CH

    pltpu.sync_copy(uids_hbm.at[pl.ds(b0, CH)], idx_v)
    pltpu.async_copy(x3a_hbm.at[idx_v], rows_v, sem).wait()
    pltpu.sync_copy(rows_v, ugl_hbm.at[pl.ds(b0, CH), :])
    pltpu.async_copy(x3b_hbm.at[idx_v], rows_v, sem).wait()
    pltpu.sync_copy(rows_v, ugr_hbm.at[pl.ds(b0, CH), :])

    pltpu.sync_copy(iids_hbm.at[pl.ds(b0, CH)], idx_v)

    @pl.loop(0, CH // 16)
    def _(g):
      sl = pl.ds(g * 16, 16)
      idx_v[sl] = idx_v[sl] + NU

    pltpu.async_copy(x3a_hbm.at[idx_v], rows_v, sem).wait()
    pltpu.sync_copy(rows_v, igl_hbm.at[pl.ds(b0, CH), :])
    pltpu.async_copy(x3b_hbm.at[idx_v], rows_v, sem).wait()
    pltpu.sync_copy(rows_v, igr_hbm.at[pl.ds(b0, CH), :])

  return k(user_ids, item_ids, x3a, x3b)


def kernel(user_ids, item_ids, content_features, edge_index, user_emb_table,
           item_emb_table, W_content, b_content, W1, att_src1, att_dst1, b1,
           W2, att_src2, att_dst2, b2, Wp1, bp1, Wp2, bp2):
  user_ids = user_ids.astype(jnp.int32)
  item_ids = item_ids.astype(jnp.int32)
  src = edge_index[0].astype(jnp.int32)
  dst = edge_index[1].astype(jnp.int32)

  c_emb = _tc_content(content_features, W_content.T,
                      b_content.reshape(1, 32))

  xL, xR, u_emb = _sc_build_x(user_ids, item_ids, c_emb,
                              user_emb_table, item_emb_table)

  W1st = jnp.stack([W1[:, :32], W1[:, 32:]])
  hh1p = _tc_hmatmul(xL.reshape(XP, 128), xR.reshape(XP, 128), W1st)
  hh1 = hh1p.reshape(2 * XROWS, 32)

  x2L, x2R, _, _ = _sc_edges1(src, dst, hh1, att_src1.reshape(64),
                              att_dst1.reshape(64), b1)

  h2p = _tc_hmatmul(x2L.reshape(XP, 128), x2R.reshape(XP, 128),
                    W2.reshape(1, 64, 32))
  h2 = h2p.reshape(XROWS, 32)

  x3a, x3b, _, _, _ = _sc_edges2(src, dst, h2, att_src2.reshape(32),
                                 att_dst2.reshape(32), b2)

  ugL, ugR, igL, igR = _sc_gather_out(user_ids, item_ids, x3a, x3b)

  out = _tc_mlp(u_emb, igL, igR, ugL, ugR, Wp1.T, bp1.reshape(1, 32),
                Wp2.T, bp2.reshape(1, 1))
  return out.reshape(B)
